# baseline (speedup 1.0000x reference)
import functools

import jax
import jax.numpy as jnp
from jax import lax
from jax.experimental import pallas as pl
from jax.experimental.pallas import tpu as pltpu
from jax.experimental.pallas import tpu_sc as plsc

N = 16384
B = 8
S = 128
KNB = 16
FD = 128
TD = 768
INF = 1e10


def _gelu(x):
    return x * 0.5 * (1.0 + jax.lax.erf(x * 0.7071067811865476))


def _mlp1_body(x_ref, w0, b0, w1, b1, w2, b2, w3, b3, o_ref):
    h = _gelu(jnp.dot(x_ref[...], w0[...], preferred_element_type=jnp.float32) + b0[...])
    h = _gelu(jnp.dot(h, w1[...], preferred_element_type=jnp.float32) + b1[...])
    h = _gelu(jnp.dot(h, w2[...], preferred_element_type=jnp.float32) + b2[...])
    o_ref[...] = jnp.dot(h, w3[...], preferred_element_type=jnp.float32) + b3[...]


def _run_mlp1(features, ws, bs):
    blk = 2048
    grid = N // blk
    full = lambda shape: pl.BlockSpec(shape, lambda i: (0,) * len(shape))
    in_specs = [pl.BlockSpec((blk, FD), lambda i: (i, 0))]
    for w, b in zip(ws, bs):
        in_specs.append(full(w.shape))
        in_specs.append(full((1,) + b.shape))
    args = [features]
    for w, b in zip(ws, bs):
        args.append(w)
        args.append(b.reshape(1, -1))
    return pl.pallas_call(
        _mlp1_body,
        grid=(grid,),
        in_specs=in_specs,
        out_specs=pl.BlockSpec((blk, TD), lambda i: (i, 0)),
        out_shape=jax.ShapeDtypeStruct((N, TD), jnp.float32),
    )(*args)


def _fps_body(xT_ref, p4_ref, bid_ref, cx_ref, cy_ref, cz_ref, ct_ref, gi_ref):
    xr = xT_ref[0:1, :]
    yr = xT_ref[1:2, :]
    zr = xT_ref[2:3, :]
    tr = xT_ref[3:4, :]
    bid = bid_ref[0:1, :]
    brow = jax.lax.broadcasted_iota(jnp.int32, (B, 1), 0)
    maskB = bid == brow
    gidx = jax.lax.broadcasted_iota(jnp.int32, (B, N), 1)
    lane = jax.lax.broadcasted_iota(jnp.int32, (B, S), 1)
    mind0 = jnp.where(maskB, jnp.float32(INF), jnp.float32(-INF))
    start = jnp.min(jnp.where(maskB, gidx, N), axis=1, keepdims=True)
    zf = jnp.zeros((B, S), jnp.float32)
    zi = jnp.zeros((B, S), jnp.int32)

    def body(s, carry):
        mind, cur, ax, ay, az, at, ai = carry
        onehot = (gidx == cur).astype(jnp.float32)
        cp = jnp.dot(onehot, p4_ref[...], precision=jax.lax.Precision.HIGHEST,
                     preferred_element_type=jnp.float32)
        cpx = cp[:, 0:1]
        cpy = cp[:, 1:2]
        cpz = cp[:, 2:3]
        cpt = cp[:, 3:4]
        sl = lane == s
        ax = ax + jnp.where(sl, cpx, 0.0)
        ay = ay + jnp.where(sl, cpy, 0.0)
        az = az + jnp.where(sl, cpz, 0.0)
        at = at + jnp.where(sl, cpt, 0.0)
        ai = ai + jnp.where(sl, cur, 0)
        dx = xr - cpx
        d = dx * dx
        dy = yr - cpy
        d = d + dy * dy
        dz = zr - cpz
        d = d + dz * dz
        dt = tr - cpt
        d = d + dt * dt
        mind = jnp.where(maskB, jnp.minimum(mind, d), jnp.float32(-INF))
        m = jnp.max(mind, axis=1, keepdims=True)
        nxt = jnp.min(jnp.where(mind == m, gidx, N), axis=1, keepdims=True)
        return mind, nxt, ax, ay, az, at, ai

    _, _, ax, ay, az, at, ai = jax.lax.fori_loop(
        0, S, body, (mind0, start, zf, zf, zf, zf, zi))
    cx_ref[...] = ax
    cy_ref[...] = ay
    cz_ref[...] = az
    ct_ref[...] = at
    gi_ref[...] = ai


def _run_fps(xT, p4, bid2):
    full = lambda shape: pl.BlockSpec(shape, lambda: (0,) * len(shape))
    outs = [jax.ShapeDtypeStruct((B, S), jnp.float32)] * 4 + [
        jax.ShapeDtypeStruct((B, S), jnp.int32)]
    return pl.pallas_call(
        _fps_body,
        in_specs=[full((4, N)), full((N, 4)), full((1, N))],
        out_specs=[full((B, S))] * 5,
        out_shape=outs,
    )(xT, p4, bid2)


def _knn_body(xT_ref, bid_ref, cx_ref, cy_ref, cz_ref, ct_ref, out_ref):
    b = pl.program_id(0)
    cxb = cx_ref[0, :, :]
    cyb = cy_ref[0, :, :]
    czb = cz_ref[0, :, :]
    ctb = ct_ref[0, :, :]
    xr = xT_ref[0:1, :]
    yr = xT_ref[1:2, :]
    zr = xT_ref[2:3, :]
    tr = xT_ref[3:4, :]
    maskb = bid_ref[0:1, :] == b
    dx = xr - cxb
    d = dx * dx
    dy = yr - cyb
    d = d + dy * dy
    dz = zr - czb
    d = d + dz * dz
    dt = tr - ctb
    d = d + dt * dt
    d = jnp.where(maskb, d, jnp.float32(INF))
    gidx = jax.lax.broadcasted_iota(jnp.int32, (S, N), 1)
    for j in range(KNB):
        m = jnp.min(d, axis=1, keepdims=True)
        ij = jnp.min(jnp.where(d == m, gidx, N), axis=1, keepdims=True)
        out_ref[0, :, j:j + 1] = ij
        d = jnp.where(gidx == ij, jnp.float32(INF), d)


def _run_knn(xT, bid2, cx3, cy3, cz3, ct3):
    full = lambda shape: pl.BlockSpec(shape, lambda b: (0,) * len(shape))
    cspec = pl.BlockSpec((1, S, 1), lambda b: (b, 0, 0))
    return pl.pallas_call(
        _knn_body,
        grid=(B,),
        in_specs=[full((4, N)), full((1, N)), cspec, cspec, cspec, cspec],
        out_specs=pl.BlockSpec((1, S, KNB), lambda b: (b, 0, 0)),
        out_shape=jax.ShapeDtypeStruct((B, S, KNB), jnp.int32),
    )(xT, bid2, cx3, cy3, cz3, ct3)


def _pool_body(pf_ref, knn_ref, cnt_ref, off_ref, tok_ref):
    for b in range(B):
        cnt = cnt_ref[b]
        off = off_ref[b]
        small = cnt <= S

        @pl.when(small)
        def _():
            def inner(s, _):
                idx = jnp.minimum(off + s, N - 1)
                tok_ref[pl.ds(b * S + s, 1), :] = pf_ref[pl.ds(idx, 1), :]
                return 0
            jax.lax.fori_loop(0, S, inner, 0)

        @pl.when(jnp.logical_not(small))
        def _():
            def inner(s, _):
                acc = pf_ref[pl.ds(knn_ref[b, s, 0], 1), :]
                for j in range(1, KNB):
                    acc = jnp.maximum(acc, pf_ref[pl.ds(knn_ref[b, s, j], 1), :])
                tok_ref[pl.ds(b * S + s, 1), :] = acc
                return 0
            jax.lax.fori_loop(0, S, inner, 0)


def _run_pool(pf, knn, counts, offsets):
    full = lambda shape: pl.BlockSpec(shape, lambda: (0,) * len(shape))
    smem = pl.BlockSpec(memory_space=pltpu.SMEM)
    return pl.pallas_call(
        _pool_body,
        in_specs=[full((N, TD)), smem, smem, smem],
        out_specs=full((B * S, TD)),
        out_shape=jax.ShapeDtypeStruct((B * S, TD), jnp.float32),
    )(pf, knn, counts, offsets)


NPAD = N + 16


def _sc_scalar(ref, i):
    v = plsc.load_gather(ref, [jnp.full((16,), i, jnp.int32)])
    return v[0]


def _sc_knn_pool(xT, p4, pf, cx, cy, cz, ct, gi, counts, offsets):
    mesh = plsc.VectorSubcoreMesh(core_axis_name="c", subcore_axis_name="s",
                                  num_cores=2)

    @functools.partial(
        pl.kernel,
        out_type=[jax.ShapeDtypeStruct((B * S, TD), jnp.float32),
                  jax.ShapeDtypeStruct((4, B * S), jnp.float32)],
        mesh=mesh,
        compiler_params=pltpu.CompilerParams(needs_layout_passes=False),
        scratch_types=[
            pltpu.VMEM((4, NPAD), jnp.float32),
            pltpu.VMEM((32,), jnp.float32),
            pltpu.VMEM((32,), jnp.float32),
            pltpu.VMEM((32,), jnp.float32),
            pltpu.VMEM((32,), jnp.float32),
            pltpu.VMEM((16,), jnp.int32),
            pltpu.VMEM((16,), jnp.int32),
            pltpu.VMEM((16,), jnp.int32),
            pltpu.VMEM((32,), jnp.int32),
            pltpu.VMEM((KNB, TD), jnp.float32),
            pltpu.VMEM((32, TD), jnp.float32),
            pltpu.VMEM((4, 32), jnp.float32),
            pltpu.SemaphoreType.DMA,
        ],
    )
    def body(xT_h, p4_h, pf_h, cx_h, cy_h, cz_h, ct_h, gi_h, cnt_h, off_h,
             tok_h, cen_h,
             coords_v, cxv, cyv, czv, ctv, cntv, offv, idxv, idx32v,
             rows_v, out_v, cenr_v, sem):
        cid = lax.axis_index("c")
        sid = lax.axis_index("s")
        wid = sid * 2 + cid
        b = wid // 4
        q = wid % 4
        s0 = q * 32
        lane = lax.broadcasted_iota(jnp.int32, (16,), 0)

        pltpu.sync_copy(cnt_h, cntv)
        pltpu.sync_copy(off_h, offv)
        cnt = _sc_scalar(cntv, b)
        off = _sc_scalar(offv, b)
        small = cnt <= S
        pltpu.sync_copy(xT_h, coords_v.at[:, :N])

        @pl.when(jnp.logical_not(small))
        def _():
            pltpu.sync_copy(gi_h.at[b, pl.ds(s0, 32)], idx32v)
            pltpu.sync_copy(cx_h.at[b, pl.ds(s0, 32)], cxv)
            pltpu.sync_copy(cy_h.at[b, pl.ds(s0, 32)], cyv)
            pltpu.sync_copy(cz_h.at[b, pl.ds(s0, 32)], czv)
            pltpu.sync_copy(ct_h.at[b, pl.ds(s0, 32)], ctv)
            end = off + cnt
            abase = pl.multiple_of((off // 16) * 16, 16)
            nblk = (end - abase + 15) // 16

            def tok_body(sl, _):
                cxs = _sc_scalar(cxv, sl)
                cys = _sc_scalar(cyv, sl)
                czs = _sc_scalar(czv, sl)
                cts = _sc_scalar(ctv, sl)

                def blk(i, carry):
                    tv, ti = carry
                    base = pl.multiple_of(abase + i * 16, 16)
                    dx = coords_v[0, pl.ds(base, 16)] - cxs
                    d = dx * dx
                    dy = coords_v[1, pl.ds(base, 16)] - cys
                    d = d + dy * dy
                    dz = coords_v[2, pl.ds(base, 16)] - czs
                    d = d + dz * dz
                    dt = coords_v[3, pl.ds(base, 16)] - cts
                    d = d + dt * dt
                    iv = base + lane
                    d = jnp.where((iv >= off) & (iv < end), d,
                                  jnp.float32(INF))
                    sd, si = plsc.sort_key_val(d, iv)
                    rd = lax.rev(sd, (0,))
                    ri = lax.rev(si, (0,))
                    keep_new = rd < tv
                    mv = jnp.where(keep_new, rd, tv)
                    mi = jnp.where(keep_new, ri, ti)
                    return tuple(plsc.sort_key_val(mv, mi))

                tv0 = jnp.full((16,), jnp.float32(INF))
                ti0 = jnp.zeros((16,), jnp.int32)
                _, ti = lax.fori_loop(0, nblk, blk, (tv0, ti0))
                idxv[...] = ti
                pltpu.async_copy(pf_h.at[idxv], rows_v, sem).wait()

                def pool(c, _):
                    cb = pl.multiple_of(c * 16, 16)
                    acc = rows_v[0, pl.ds(cb, 16)]
                    for r in range(1, KNB):
                        acc = jnp.maximum(acc, rows_v[r, pl.ds(cb, 16)])
                    out_v[sl, pl.ds(cb, 16)] = acc
                    return 0

                lax.fori_loop(0, TD // 16, pool, 0)
                return 0

            lax.fori_loop(0, 32, tok_body, 0)

        @pl.when(small)
        def _():
            base = off + s0
            idx32v[pl.ds(0, 16)] = jnp.minimum(base + lane, N - 1)
            idx32v[pl.ds(16, 16)] = jnp.minimum(base + 16 + lane, N - 1)
            pltpu.async_copy(pf_h.at[idx32v], out_v, sem).wait()

        t0 = b * S + s0
        for h in (0, 16):
            idx16 = idx32v[pl.ds(h, 16)]
            for c in range(4):
                g = plsc.load_gather(coords_v,
                                     [jnp.full((16,), c, jnp.int32), idx16])
                cenr_v[c, pl.ds(h, 16)] = g
        for c in range(4):
            pltpu.sync_copy(cenr_v.at[c, pl.ds(0, 32)],
                            cen_h.at[c, pl.ds(t0, 32)])
        pltpu.sync_copy(out_v, tok_h.at[pl.ds(t0, 32), :])

    return body(xT, p4, pf, cx, cy, cz, ct, gi, counts, offsets)


def _mlp2_body(cnt_ref, tok_ref, cen_ref, w0, b0, w1, b1,
               tokens_ref, cents_ref, valid_ref):
    b = pl.program_id(0)
    cnt = cnt_ref[b]
    small = cnt <= S
    t = tok_ref[...]
    h = _gelu(jnp.dot(t, w0[...], preferred_element_type=jnp.float32) + b0[...])
    h = jnp.dot(h, w1[...], preferred_element_type=jnp.float32) + b1[...]
    lim = jnp.where(small, jnp.minimum(cnt, S), S)
    sidx = jax.lax.broadcasted_iota(jnp.int32, (S, 1), 0)
    validc = sidx < lim
    lidx = jax.lax.broadcasted_iota(jnp.int32, (1, S), 1)
    validr = lidx < lim
    tokens_ref[0, :, :] = jnp.where(validc, h, 0.0)
    cents_ref[0, :, :] = jnp.where(validc, cen_ref[...], 0.0)
    valid_ref[0, :, :] = validr


def _run_mlp2(counts, tok, cen, w0, b0, w1, b1):
    full = lambda shape: pl.BlockSpec(shape, lambda b: (0,) * len(shape))
    smem = pl.BlockSpec(memory_space=pltpu.SMEM)
    return pl.pallas_call(
        _mlp2_body,
        grid=(B,),
        in_specs=[smem,
                  pl.BlockSpec((S, TD), lambda b: (b, 0)),
                  pl.BlockSpec((S, 4), lambda b: (b, 0)),
                  full((TD, TD)), full((1, TD)), full((TD, TD)), full((1, TD))],
        out_specs=[pl.BlockSpec((1, S, TD), lambda b: (b, 0, 0)),
                   pl.BlockSpec((1, S, 4), lambda b: (b, 0, 0)),
                   pl.BlockSpec((1, 1, S), lambda b: (b, 0, 0))],
        out_shape=[jax.ShapeDtypeStruct((B, S, TD), jnp.float32),
                   jax.ShapeDtypeStruct((B, S, 4), jnp.float32),
                   jax.ShapeDtypeStruct((B, 1, S), jnp.bool_)],
    )(counts, tok, cen, w0, b0.reshape(1, -1), w1, b1.reshape(1, -1))


def kernel(coords, features, batch_ids, times,
           W1_0, b1_0, W1_1, b1_1, W1_2, b1_2, W1_3, b1_3,
           W2_0, b2_0, W2_1, b2_1):
    bid = batch_ids.astype(jnp.int32)
    counts = jnp.bincount(bid, length=B).astype(jnp.int32)
    offsets = (jnp.cumsum(counts) - counts).astype(jnp.int32)
    p4 = jnp.concatenate([coords[:, :3], times], axis=1)
    xT = p4.T
    bid2 = bid.reshape(1, N)

    pf = _run_mlp1(features, [W1_0, W1_1, W1_2, W1_3], [b1_0, b1_1, b1_2, b1_3])
    cx, cy, cz, ct, gi = _run_fps(xT, p4, bid2)
    cnt16 = jnp.zeros((16,), jnp.int32).at[:B].set(counts)
    off16 = jnp.zeros((16,), jnp.int32).at[:B].set(offsets)
    tok, cenT = _sc_knn_pool(xT, p4, pf, cx, cy, cz, ct, gi, cnt16, off16)
    tokens, centroids, valid3 = _run_mlp2(counts, tok, cenT.T,
                                          W2_0, b2_0, W2_1, b2_1)
    return tokens, centroids, valid3.reshape(B, S)


# speedup vs baseline: 1.7277x; 1.7277x over previous
import functools

import jax
import jax.numpy as jnp
from jax import lax
from jax.experimental import pallas as pl
from jax.experimental.pallas import tpu as pltpu
from jax.experimental.pallas import tpu_sc as plsc

N = 16384
B = 8
S = 128
KNB = 16
FD = 128
TD = 768
INF = 1e10


def _gelu(x):
    return x * 0.5 * (1.0 + jax.lax.erf(x * 0.7071067811865476))


def _mlp1_body(x_ref, w0, b0, w1, b1, w2, b2, w3, b3, o_ref):
    h = _gelu(jnp.dot(x_ref[...], w0[...], preferred_element_type=jnp.float32) + b0[...])
    h = _gelu(jnp.dot(h, w1[...], preferred_element_type=jnp.float32) + b1[...])
    h = _gelu(jnp.dot(h, w2[...], preferred_element_type=jnp.float32) + b2[...])
    o_ref[...] = jnp.dot(h, w3[...], preferred_element_type=jnp.float32) + b3[...]


def _run_mlp1(features, ws, bs):
    blk = 2048
    grid = N // blk
    full = lambda shape: pl.BlockSpec(shape, lambda i: (0,) * len(shape))
    in_specs = [pl.BlockSpec((blk, FD), lambda i: (i, 0))]
    for w, b in zip(ws, bs):
        in_specs.append(full(w.shape))
        in_specs.append(full((1,) + b.shape))
    args = [features]
    for w, b in zip(ws, bs):
        args.append(w)
        args.append(b.reshape(1, -1))
    return pl.pallas_call(
        _mlp1_body,
        grid=(grid,),
        in_specs=in_specs,
        out_specs=pl.BlockSpec((blk, TD), lambda i: (i, 0)),
        out_shape=jax.ShapeDtypeStruct((N, TD), jnp.float32),
    )(*args)


def _fps_body(xT_ref, p4_ref, bid_ref, cx_ref, cy_ref, cz_ref, ct_ref, gi_ref):
    xr = xT_ref[0:1, :]
    yr = xT_ref[1:2, :]
    zr = xT_ref[2:3, :]
    tr = xT_ref[3:4, :]
    bid = bid_ref[0:1, :]
    brow = jax.lax.broadcasted_iota(jnp.int32, (B, 1), 0)
    maskB = bid == brow
    gidx = jax.lax.broadcasted_iota(jnp.int32, (B, N), 1)
    lane = jax.lax.broadcasted_iota(jnp.int32, (B, S), 1)
    mind0 = jnp.where(maskB, jnp.float32(INF), jnp.float32(-INF))
    start = jnp.min(jnp.where(maskB, gidx, N), axis=1, keepdims=True)
    zf = jnp.zeros((B, S), jnp.float32)
    zi = jnp.zeros((B, S), jnp.int32)

    def body(s, carry):
        mind, cur, ax, ay, az, at, ai = carry
        onehot = gidx == cur
        cpx = jnp.sum(jnp.where(onehot, xr, 0.0), axis=1, keepdims=True)
        cpy = jnp.sum(jnp.where(onehot, yr, 0.0), axis=1, keepdims=True)
        cpz = jnp.sum(jnp.where(onehot, zr, 0.0), axis=1, keepdims=True)
        cpt = jnp.sum(jnp.where(onehot, tr, 0.0), axis=1, keepdims=True)
        sl = lane == s
        ax = ax + jnp.where(sl, cpx, 0.0)
        ay = ay + jnp.where(sl, cpy, 0.0)
        az = az + jnp.where(sl, cpz, 0.0)
        at = at + jnp.where(sl, cpt, 0.0)
        ai = ai + jnp.where(sl, cur, 0)
        dx = xr - cpx
        d = dx * dx
        dy = yr - cpy
        d = d + dy * dy
        dz = zr - cpz
        d = d + dz * dz
        dt = tr - cpt
        d = d + dt * dt
        mind = jnp.minimum(mind, d)
        m = jnp.max(mind, axis=1, keepdims=True)
        nxt = jnp.min(jnp.where(mind == m, gidx, N), axis=1, keepdims=True)
        return mind, nxt, ax, ay, az, at, ai

    _, _, ax, ay, az, at, ai = jax.lax.fori_loop(
        0, S, body, (mind0, start, zf, zf, zf, zf, zi))
    cx_ref[...] = ax
    cy_ref[...] = ay
    cz_ref[...] = az
    ct_ref[...] = at
    gi_ref[...] = ai


def _run_fps(xT, p4, bid2):
    full = lambda shape: pl.BlockSpec(shape, lambda: (0,) * len(shape))
    outs = [jax.ShapeDtypeStruct((B, S), jnp.float32)] * 4 + [
        jax.ShapeDtypeStruct((B, S), jnp.int32)]
    return pl.pallas_call(
        _fps_body,
        in_specs=[full((4, N)), full((N, 4)), full((1, N))],
        out_specs=[full((B, S))] * 5,
        out_shape=outs,
    )(xT, p4, bid2)


def _knn_body(xT_ref, bid_ref, cx_ref, cy_ref, cz_ref, ct_ref, out_ref):
    b = pl.program_id(0)
    cxb = cx_ref[0, :, :]
    cyb = cy_ref[0, :, :]
    czb = cz_ref[0, :, :]
    ctb = ct_ref[0, :, :]
    xr = xT_ref[0:1, :]
    yr = xT_ref[1:2, :]
    zr = xT_ref[2:3, :]
    tr = xT_ref[3:4, :]
    maskb = bid_ref[0:1, :] == b
    dx = xr - cxb
    d = dx * dx
    dy = yr - cyb
    d = d + dy * dy
    dz = zr - czb
    d = d + dz * dz
    dt = tr - ctb
    d = d + dt * dt
    d = jnp.where(maskb, d, jnp.float32(INF))
    gidx = jax.lax.broadcasted_iota(jnp.int32, (S, N), 1)
    for j in range(KNB):
        m = jnp.min(d, axis=1, keepdims=True)
        ij = jnp.min(jnp.where(d == m, gidx, N), axis=1, keepdims=True)
        out_ref[0, :, j:j + 1] = ij
        d = jnp.where(gidx == ij, jnp.float32(INF), d)


def _run_knn(xT, bid2, cx3, cy3, cz3, ct3):
    full = lambda shape: pl.BlockSpec(shape, lambda b: (0,) * len(shape))
    cspec = pl.BlockSpec((1, S, 1), lambda b: (b, 0, 0))
    return pl.pallas_call(
        _knn_body,
        grid=(B,),
        in_specs=[full((4, N)), full((1, N)), cspec, cspec, cspec, cspec],
        out_specs=pl.BlockSpec((1, S, KNB), lambda b: (b, 0, 0)),
        out_shape=jax.ShapeDtypeStruct((B, S, KNB), jnp.int32),
    )(xT, bid2, cx3, cy3, cz3, ct3)


def _pool_body(pf_ref, knn_ref, cnt_ref, off_ref, tok_ref):
    for b in range(B):
        cnt = cnt_ref[b]
        off = off_ref[b]
        small = cnt <= S

        @pl.when(small)
        def _():
            def inner(s, _):
                idx = jnp.minimum(off + s, N - 1)
                tok_ref[pl.ds(b * S + s, 1), :] = pf_ref[pl.ds(idx, 1), :]
                return 0
            jax.lax.fori_loop(0, S, inner, 0)

        @pl.when(jnp.logical_not(small))
        def _():
            def inner(s, _):
                acc = pf_ref[pl.ds(knn_ref[b, s, 0], 1), :]
                for j in range(1, KNB):
                    acc = jnp.maximum(acc, pf_ref[pl.ds(knn_ref[b, s, j], 1), :])
                tok_ref[pl.ds(b * S + s, 1), :] = acc
                return 0
            jax.lax.fori_loop(0, S, inner, 0)


def _run_pool(pf, knn, counts, offsets):
    full = lambda shape: pl.BlockSpec(shape, lambda: (0,) * len(shape))
    smem = pl.BlockSpec(memory_space=pltpu.SMEM)
    return pl.pallas_call(
        _pool_body,
        in_specs=[full((N, TD)), smem, smem, smem],
        out_specs=full((B * S, TD)),
        out_shape=jax.ShapeDtypeStruct((B * S, TD), jnp.float32),
    )(pf, knn, counts, offsets)


NPAD = N + 16


def _sc_scalar(ref, i):
    v = plsc.load_gather(ref, [jnp.full((16,), i, jnp.int32)])
    return v[0]


def _sc_knn_pool(xT, p4, pf, cx, cy, cz, ct, gi, counts, offsets):
    mesh = plsc.VectorSubcoreMesh(core_axis_name="c", subcore_axis_name="s",
                                  num_cores=2)

    @functools.partial(
        pl.kernel,
        out_type=[jax.ShapeDtypeStruct((B * S, TD), jnp.float32),
                  jax.ShapeDtypeStruct((4, B * S), jnp.float32)],
        mesh=mesh,
        compiler_params=pltpu.CompilerParams(needs_layout_passes=False),
        scratch_types=[
            pltpu.VMEM((4, NPAD), jnp.float32),
            pltpu.VMEM((32,), jnp.float32),
            pltpu.VMEM((32,), jnp.float32),
            pltpu.VMEM((32,), jnp.float32),
            pltpu.VMEM((32,), jnp.float32),
            pltpu.VMEM((16,), jnp.int32),
            pltpu.VMEM((16,), jnp.int32),
            pltpu.VMEM((16,), jnp.int32),
            pltpu.VMEM((32,), jnp.int32),
            pltpu.VMEM((KNB, TD), jnp.float32),
            pltpu.VMEM((32, TD), jnp.float32),
            pltpu.VMEM((4, 32), jnp.float32),
            pltpu.SemaphoreType.DMA,
        ],
    )
    def body(xT_h, p4_h, pf_h, cx_h, cy_h, cz_h, ct_h, gi_h, cnt_h, off_h,
             tok_h, cen_h,
             coords_v, cxv, cyv, czv, ctv, cntv, offv, idxv, idx32v,
             rows_v, out_v, cenr_v, sem):
        cid = lax.axis_index("c")
        sid = lax.axis_index("s")
        wid = sid * 2 + cid
        b = wid // 4
        q = wid % 4
        s0 = q * 32
        lane = lax.broadcasted_iota(jnp.int32, (16,), 0)

        pltpu.sync_copy(cnt_h, cntv)
        pltpu.sync_copy(off_h, offv)
        cnt = _sc_scalar(cntv, b)
        off = _sc_scalar(offv, b)
        small = cnt <= S
        pltpu.sync_copy(xT_h, coords_v.at[:, :N])

        @pl.when(jnp.logical_not(small))
        def _():
            pltpu.sync_copy(gi_h.at[b, pl.ds(s0, 32)], idx32v)
            pltpu.sync_copy(cx_h.at[b, pl.ds(s0, 32)], cxv)
            pltpu.sync_copy(cy_h.at[b, pl.ds(s0, 32)], cyv)
            pltpu.sync_copy(cz_h.at[b, pl.ds(s0, 32)], czv)
            pltpu.sync_copy(ct_h.at[b, pl.ds(s0, 32)], ctv)
            end = off + cnt
            abase = pl.multiple_of((off // 16) * 16, 16)
            nblk = (end - abase + 15) // 16

            def tok_body(sl, _):
                cxs = _sc_scalar(cxv, sl)
                cys = _sc_scalar(cyv, sl)
                czs = _sc_scalar(czv, sl)
                cts = _sc_scalar(ctv, sl)

                def blk(i, carry):
                    tv, ti = carry
                    base = pl.multiple_of(abase + i * 16, 16)
                    dx = coords_v[0, pl.ds(base, 16)] - cxs
                    d = dx * dx
                    dy = coords_v[1, pl.ds(base, 16)] - cys
                    d = d + dy * dy
                    dz = coords_v[2, pl.ds(base, 16)] - czs
                    d = d + dz * dz
                    dt = coords_v[3, pl.ds(base, 16)] - cts
                    d = d + dt * dt
                    iv = base + lane
                    d = jnp.where((iv >= off) & (iv < end), d,
                                  jnp.float32(INF))
                    hit = plsc.all_reduce_population_count(d < tv[15])[0] > 0

                    def merge(args):
                        tv, ti, d, iv = args
                        sd, si = plsc.sort_key_val(d, iv)
                        rd = lax.rev(sd, (0,))
                        ri = lax.rev(si, (0,))
                        keep_new = rd < tv
                        mv = jnp.where(keep_new, rd, tv)
                        mi = jnp.where(keep_new, ri, ti)
                        return tuple(plsc.sort_key_val(mv, mi))

                    def keep(args):
                        return args[0], args[1]

                    return lax.cond(hit, merge, keep, (tv, ti, d, iv))

                tv0 = jnp.full((16,), jnp.float32(INF))
                ti0 = jnp.zeros((16,), jnp.int32)
                _, ti = lax.fori_loop(0, nblk, blk, (tv0, ti0))
                idxv[...] = ti
                pltpu.async_copy(pf_h.at[idxv], rows_v, sem).wait()

                def pool(c, _):
                    cb = pl.multiple_of(c * 16, 16)
                    acc = rows_v[0, pl.ds(cb, 16)]
                    for r in range(1, KNB):
                        acc = jnp.maximum(acc, rows_v[r, pl.ds(cb, 16)])
                    out_v[sl, pl.ds(cb, 16)] = acc
                    return 0

                lax.fori_loop(0, TD // 16, pool, 0)
                return 0

            lax.fori_loop(0, 32, tok_body, 0)

        @pl.when(small)
        def _():
            base = off + s0
            idx32v[pl.ds(0, 16)] = jnp.minimum(base + lane, N - 1)
            idx32v[pl.ds(16, 16)] = jnp.minimum(base + 16 + lane, N - 1)
            pltpu.async_copy(pf_h.at[idx32v], out_v, sem).wait()

        t0 = b * S + s0
        for h in (0, 16):
            idx16 = idx32v[pl.ds(h, 16)]
            for c in range(4):
                g = plsc.load_gather(coords_v,
                                     [jnp.full((16,), c, jnp.int32), idx16])
                cenr_v[c, pl.ds(h, 16)] = g
        for c in range(4):
            pltpu.sync_copy(cenr_v.at[c, pl.ds(0, 32)],
                            cen_h.at[c, pl.ds(t0, 32)])
        pltpu.sync_copy(out_v, tok_h.at[pl.ds(t0, 32), :])

    return body(xT, p4, pf, cx, cy, cz, ct, gi, counts, offsets)


def _mlp2_body(cnt_ref, tok_ref, cen_ref, w0, b0, w1, b1,
               tokens_ref, cents_ref, valid_ref):
    b = pl.program_id(0)
    cnt = cnt_ref[b]
    small = cnt <= S
    t = tok_ref[...]
    h = _gelu(jnp.dot(t, w0[...], preferred_element_type=jnp.float32) + b0[...])
    h = jnp.dot(h, w1[...], preferred_element_type=jnp.float32) + b1[...]
    lim = jnp.where(small, jnp.minimum(cnt, S), S)
    sidx = jax.lax.broadcasted_iota(jnp.int32, (S, 1), 0)
    validc = sidx < lim
    lidx = jax.lax.broadcasted_iota(jnp.int32, (1, S), 1)
    validr = lidx < lim
    tokens_ref[0, :, :] = jnp.where(validc, h, 0.0)
    cents_ref[0, :, :] = jnp.where(validc, cen_ref[...], 0.0)
    valid_ref[0, :, :] = validr


def _run_mlp2(counts, tok, cen, w0, b0, w1, b1):
    full = lambda shape: pl.BlockSpec(shape, lambda b: (0,) * len(shape))
    smem = pl.BlockSpec(memory_space=pltpu.SMEM)
    return pl.pallas_call(
        _mlp2_body,
        grid=(B,),
        in_specs=[smem,
                  pl.BlockSpec((S, TD), lambda b: (b, 0)),
                  pl.BlockSpec((S, 4), lambda b: (b, 0)),
                  full((TD, TD)), full((1, TD)), full((TD, TD)), full((1, TD))],
        out_specs=[pl.BlockSpec((1, S, TD), lambda b: (b, 0, 0)),
                   pl.BlockSpec((1, S, 4), lambda b: (b, 0, 0)),
                   pl.BlockSpec((1, 1, S), lambda b: (b, 0, 0))],
        out_shape=[jax.ShapeDtypeStruct((B, S, TD), jnp.float32),
                   jax.ShapeDtypeStruct((B, S, 4), jnp.float32),
                   jax.ShapeDtypeStruct((B, 1, S), jnp.bool_)],
    )(counts, tok, cen, w0, b0.reshape(1, -1), w1, b1.reshape(1, -1))


def kernel(coords, features, batch_ids, times,
           W1_0, b1_0, W1_1, b1_1, W1_2, b1_2, W1_3, b1_3,
           W2_0, b2_0, W2_1, b2_1):
    bid = batch_ids.astype(jnp.int32)
    counts = jnp.bincount(bid, length=B).astype(jnp.int32)
    offsets = (jnp.cumsum(counts) - counts).astype(jnp.int32)
    p4 = jnp.concatenate([coords[:, :3], times], axis=1)
    xT = p4.T
    bid2 = bid.reshape(1, N)

    pf = _run_mlp1(features, [W1_0, W1_1, W1_2, W1_3], [b1_0, b1_1, b1_2, b1_3])
    cx, cy, cz, ct, gi = _run_fps(xT, p4, bid2)
    cnt16 = jnp.zeros((16,), jnp.int32).at[:B].set(counts)
    off16 = jnp.zeros((16,), jnp.int32).at[:B].set(offsets)
    tok, cenT = _sc_knn_pool(xT, p4, pf, cx, cy, cz, ct, gi, cnt16, off16)
    tokens, centroids, valid3 = _run_mlp2(counts, tok, cenT.T,
                                          W2_0, b2_0, W2_1, b2_1)
    return tokens, centroids, valid3.reshape(B, S)


# speedup vs baseline: 2.2258x; 1.2883x over previous
import functools

import jax
import jax.numpy as jnp
from jax import lax
from jax.experimental import pallas as pl
from jax.experimental.pallas import tpu as pltpu
from jax.experimental.pallas import tpu_sc as plsc

N = 16384
B = 8
S = 128
KNB = 16
FD = 128
TD = 768
INF = 1e10


def _gelu(x):
    return x * 0.5 * (1.0 + jax.lax.erf(x * 0.7071067811865476))


def _bdot(x, w):
    return jnp.dot(x.astype(jnp.bfloat16), w.astype(jnp.bfloat16),
                   preferred_element_type=jnp.float32)


def _mlp1_body(x_ref, w0, b0, w1, b1, w2, b2, w3, b3, o_ref):
    h = _gelu(_bdot(x_ref[...], w0[...]) + b0[...])
    h = _gelu(_bdot(h, w1[...]) + b1[...])
    h = _gelu(_bdot(h, w2[...]) + b2[...])
    o_ref[...] = _bdot(h, w3[...]) + b3[...]


def _run_mlp1(features, ws, bs):
    blk = 2048
    grid = N // blk
    full = lambda shape: pl.BlockSpec(shape, lambda i: (0,) * len(shape))
    in_specs = [pl.BlockSpec((blk, FD), lambda i: (i, 0))]
    for w, b in zip(ws, bs):
        in_specs.append(full(w.shape))
        in_specs.append(full((1,) + b.shape))
    args = [features]
    for w, b in zip(ws, bs):
        args.append(w)
        args.append(b.reshape(1, -1))
    return pl.pallas_call(
        _mlp1_body,
        grid=(grid,),
        in_specs=in_specs,
        out_specs=pl.BlockSpec((blk, TD), lambda i: (i, 0)),
        out_shape=jax.ShapeDtypeStruct((N, TD), jnp.float32),
    )(*args)


def _fps_body(xT_ref, p4_ref, bid_ref, cx_ref, cy_ref, cz_ref, ct_ref, gi_ref):
    xr = xT_ref[0:1, :]
    yr = xT_ref[1:2, :]
    zr = xT_ref[2:3, :]
    tr = xT_ref[3:4, :]
    bid = bid_ref[0:1, :]
    brow = jax.lax.broadcasted_iota(jnp.int32, (B, 1), 0)
    maskB = bid == brow
    gidx = jax.lax.broadcasted_iota(jnp.int32, (B, N), 1)
    lane = jax.lax.broadcasted_iota(jnp.int32, (B, S), 1)
    mind0 = jnp.where(maskB, jnp.float32(INF), jnp.float32(-INF))
    start = jnp.min(jnp.where(maskB, gidx, N), axis=1, keepdims=True)
    zf = jnp.zeros((B, S), jnp.float32)
    zi = jnp.zeros((B, S), jnp.int32)

    def body(s, carry):
        mind, cur, ax, ay, az, at, ai = carry
        onehot = gidx == cur
        cpx = jnp.sum(jnp.where(onehot, xr, 0.0), axis=1, keepdims=True)
        cpy = jnp.sum(jnp.where(onehot, yr, 0.0), axis=1, keepdims=True)
        cpz = jnp.sum(jnp.where(onehot, zr, 0.0), axis=1, keepdims=True)
        cpt = jnp.sum(jnp.where(onehot, tr, 0.0), axis=1, keepdims=True)
        sl = lane == s
        ax = ax + jnp.where(sl, cpx, 0.0)
        ay = ay + jnp.where(sl, cpy, 0.0)
        az = az + jnp.where(sl, cpz, 0.0)
        at = at + jnp.where(sl, cpt, 0.0)
        ai = ai + jnp.where(sl, cur, 0)
        dx = xr - cpx
        d = dx * dx
        dy = yr - cpy
        d = d + dy * dy
        dz = zr - cpz
        d = d + dz * dz
        dt = tr - cpt
        d = d + dt * dt
        mind = jnp.minimum(mind, d)
        m = jnp.max(mind, axis=1, keepdims=True)
        nxt = jnp.min(jnp.where(mind == m, gidx, N), axis=1, keepdims=True)
        return mind, nxt, ax, ay, az, at, ai

    _, _, ax, ay, az, at, ai = jax.lax.fori_loop(
        0, S, body, (mind0, start, zf, zf, zf, zf, zi))
    cx_ref[...] = ax
    cy_ref[...] = ay
    cz_ref[...] = az
    ct_ref[...] = at
    gi_ref[...] = ai


def _run_fps(xT, p4, bid2):
    full = lambda shape: pl.BlockSpec(shape, lambda: (0,) * len(shape))
    outs = [jax.ShapeDtypeStruct((B, S), jnp.float32)] * 4 + [
        jax.ShapeDtypeStruct((B, S), jnp.int32)]
    return pl.pallas_call(
        _fps_body,
        in_specs=[full((4, N)), full((N, 4)), full((1, N))],
        out_specs=[full((B, S))] * 5,
        out_shape=outs,
    )(xT, p4, bid2)


def _knn_body(xT_ref, bid_ref, cx_ref, cy_ref, cz_ref, ct_ref, out_ref):
    b = pl.program_id(0)
    cxb = cx_ref[0, :, :]
    cyb = cy_ref[0, :, :]
    czb = cz_ref[0, :, :]
    ctb = ct_ref[0, :, :]
    xr = xT_ref[0:1, :]
    yr = xT_ref[1:2, :]
    zr = xT_ref[2:3, :]
    tr = xT_ref[3:4, :]
    maskb = bid_ref[0:1, :] == b
    dx = xr - cxb
    d = dx * dx
    dy = yr - cyb
    d = d + dy * dy
    dz = zr - czb
    d = d + dz * dz
    dt = tr - ctb
    d = d + dt * dt
    d = jnp.where(maskb, d, jnp.float32(INF))
    gidx = jax.lax.broadcasted_iota(jnp.int32, (S, N), 1)
    for j in range(KNB):
        m = jnp.min(d, axis=1, keepdims=True)
        ij = jnp.min(jnp.where(d == m, gidx, N), axis=1, keepdims=True)
        out_ref[0, :, j:j + 1] = ij
        d = jnp.where(gidx == ij, jnp.float32(INF), d)


def _run_knn(xT, bid2, cx3, cy3, cz3, ct3):
    full = lambda shape: pl.BlockSpec(shape, lambda b: (0,) * len(shape))
    cspec = pl.BlockSpec((1, S, 1), lambda b: (b, 0, 0))
    return pl.pallas_call(
        _knn_body,
        grid=(B,),
        in_specs=[full((4, N)), full((1, N)), cspec, cspec, cspec, cspec],
        out_specs=pl.BlockSpec((1, S, KNB), lambda b: (b, 0, 0)),
        out_shape=jax.ShapeDtypeStruct((B, S, KNB), jnp.int32),
    )(xT, bid2, cx3, cy3, cz3, ct3)


def _pool_body(pf_ref, knn_ref, cnt_ref, off_ref, tok_ref):
    for b in range(B):
        cnt = cnt_ref[b]
        off = off_ref[b]
        small = cnt <= S

        @pl.when(small)
        def _():
            def inner(s, _):
                idx = jnp.minimum(off + s, N - 1)
                tok_ref[pl.ds(b * S + s, 1), :] = pf_ref[pl.ds(idx, 1), :]
                return 0
            jax.lax.fori_loop(0, S, inner, 0)

        @pl.when(jnp.logical_not(small))
        def _():
            def inner(s, _):
                acc = pf_ref[pl.ds(knn_ref[b, s, 0], 1), :]
                for j in range(1, KNB):
                    acc = jnp.maximum(acc, pf_ref[pl.ds(knn_ref[b, s, j], 1), :])
                tok_ref[pl.ds(b * S + s, 1), :] = acc
                return 0
            jax.lax.fori_loop(0, S, inner, 0)


def _run_pool(pf, knn, counts, offsets):
    full = lambda shape: pl.BlockSpec(shape, lambda: (0,) * len(shape))
    smem = pl.BlockSpec(memory_space=pltpu.SMEM)
    return pl.pallas_call(
        _pool_body,
        in_specs=[full((N, TD)), smem, smem, smem],
        out_specs=full((B * S, TD)),
        out_shape=jax.ShapeDtypeStruct((B * S, TD), jnp.float32),
    )(pf, knn, counts, offsets)


NPAD = N + 16


def _sc_scalar(ref, i):
    v = plsc.load_gather(ref, [jnp.full((16,), i, jnp.int32)])
    return v[0]


def _sc_knn_pool(xT, p4, pf, cx, cy, cz, ct, gi, counts, offsets):
    mesh = plsc.VectorSubcoreMesh(core_axis_name="c", subcore_axis_name="s",
                                  num_cores=2)

    @functools.partial(
        pl.kernel,
        out_type=[jax.ShapeDtypeStruct((B * S, TD), jnp.float32),
                  jax.ShapeDtypeStruct((4, B * S), jnp.float32)],
        mesh=mesh,
        compiler_params=pltpu.CompilerParams(needs_layout_passes=False),
        scratch_types=[
            pltpu.VMEM((4, NPAD), jnp.float32),
            pltpu.VMEM((32,), jnp.float32),
            pltpu.VMEM((32,), jnp.float32),
            pltpu.VMEM((32,), jnp.float32),
            pltpu.VMEM((32,), jnp.float32),
            pltpu.VMEM((16,), jnp.int32),
            pltpu.VMEM((16,), jnp.int32),
            pltpu.VMEM((16,), jnp.int32),
            pltpu.VMEM((32,), jnp.int32),
            pltpu.VMEM((KNB, TD), jnp.float32),
            pltpu.VMEM((32, TD), jnp.float32),
            pltpu.VMEM((4, 32), jnp.float32),
            pltpu.SemaphoreType.DMA,
        ],
    )
    def body(xT_h, p4_h, pf_h, cx_h, cy_h, cz_h, ct_h, gi_h, cnt_h, off_h,
             tok_h, cen_h,
             coords_v, cxv, cyv, czv, ctv, cntv, offv, idxv, idx32v,
             rows_v, out_v, cenr_v, sem):
        cid = lax.axis_index("c")
        sid = lax.axis_index("s")
        wid = sid * 2 + cid
        b = wid // 4
        q = wid % 4
        s0 = q * 32
        lane = lax.broadcasted_iota(jnp.int32, (16,), 0)

        pltpu.sync_copy(cnt_h, cntv)
        pltpu.sync_copy(off_h, offv)
        cnt = _sc_scalar(cntv, b)
        off = _sc_scalar(offv, b)
        small = cnt <= S
        pltpu.sync_copy(xT_h, coords_v.at[:, :N])

        @pl.when(jnp.logical_not(small))
        def _():
            pltpu.sync_copy(gi_h.at[b, pl.ds(s0, 32)], idx32v)
            pltpu.sync_copy(cx_h.at[b, pl.ds(s0, 32)], cxv)
            pltpu.sync_copy(cy_h.at[b, pl.ds(s0, 32)], cyv)
            pltpu.sync_copy(cz_h.at[b, pl.ds(s0, 32)], czv)
            pltpu.sync_copy(ct_h.at[b, pl.ds(s0, 32)], ctv)
            end = off + cnt
            abase = pl.multiple_of((off // 16) * 16, 16)
            nblk = (end - abase + 15) // 16

            def tok_body(sl, _):
                cxs = _sc_scalar(cxv, sl)
                cys = _sc_scalar(cyv, sl)
                czs = _sc_scalar(czv, sl)
                cts = _sc_scalar(ctv, sl)

                def blk(i, carry):
                    tv, ti = carry
                    base = pl.multiple_of(abase + i * 16, 16)
                    dx = coords_v[0, pl.ds(base, 16)] - cxs
                    d = dx * dx
                    dy = coords_v[1, pl.ds(base, 16)] - cys
                    d = d + dy * dy
                    dz = coords_v[2, pl.ds(base, 16)] - czs
                    d = d + dz * dz
                    dt = coords_v[3, pl.ds(base, 16)] - cts
                    d = d + dt * dt
                    iv = base + lane
                    d = jnp.where((iv >= off) & (iv < end), d,
                                  jnp.float32(INF))
                    sd, si = plsc.sort_key_val(d, iv)
                    rd = lax.rev(sd, (0,))
                    ri = lax.rev(si, (0,))
                    keep_new = rd < tv
                    mv = jnp.where(keep_new, rd, tv)
                    mi = jnp.where(keep_new, ri, ti)
                    return tuple(plsc.sort_key_val(mv, mi))

                tv0 = jnp.full((16,), jnp.float32(INF))
                ti0 = jnp.zeros((16,), jnp.int32)
                _, ti = lax.fori_loop(0, nblk, blk, (tv0, ti0))
                idxv[...] = ti
                pltpu.async_copy(pf_h.at[idxv], rows_v, sem).wait()

                def pool(c, _):
                    cb = pl.multiple_of(c * 16, 16)
                    acc = rows_v[0, pl.ds(cb, 16)]
                    for r in range(1, KNB):
                        acc = jnp.maximum(acc, rows_v[r, pl.ds(cb, 16)])
                    out_v[sl, pl.ds(cb, 16)] = acc
                    return 0

                lax.fori_loop(0, TD // 16, pool, 0)
                return 0

            lax.fori_loop(0, 32, tok_body, 0)

        @pl.when(small)
        def _():
            base = off + s0
            idx32v[pl.ds(0, 16)] = jnp.minimum(base + lane, N - 1)
            idx32v[pl.ds(16, 16)] = jnp.minimum(base + 16 + lane, N - 1)
            pltpu.async_copy(pf_h.at[idx32v], out_v, sem).wait()

        t0 = b * S + s0
        for h in (0, 16):
            idx16 = idx32v[pl.ds(h, 16)]
            for c in range(4):
                g = plsc.load_gather(coords_v,
                                     [jnp.full((16,), c, jnp.int32), idx16])
                cenr_v[c, pl.ds(h, 16)] = g
        for c in range(4):
            pltpu.sync_copy(cenr_v.at[c, pl.ds(0, 32)],
                            cen_h.at[c, pl.ds(t0, 32)])
        pltpu.sync_copy(out_v, tok_h.at[pl.ds(t0, 32), :])

    return body(xT, p4, pf, cx, cy, cz, ct, gi, counts, offsets)


def _mlp2_body(cnt_ref, tok_ref, cen_ref, w0, b0, w1, b1,
               tokens_ref, cents_ref, valid_ref):
    b = pl.program_id(0)
    cnt = cnt_ref[b]
    small = cnt <= S
    t = tok_ref[...]
    h = _gelu(_bdot(t, w0[...]) + b0[...])
    h = _bdot(h, w1[...]) + b1[...]
    lim = jnp.where(small, jnp.minimum(cnt, S), S)
    sidx = jax.lax.broadcasted_iota(jnp.int32, (S, 1), 0)
    validc = sidx < lim
    lidx = jax.lax.broadcasted_iota(jnp.int32, (1, S), 1)
    validr = lidx < lim
    tokens_ref[0, :, :] = jnp.where(validc, h, 0.0)
    cents_ref[0, :, :] = jnp.where(validc, cen_ref[...], 0.0)
    valid_ref[0, :, :] = validr


def _run_mlp2(counts, tok, cen, w0, b0, w1, b1):
    full = lambda shape: pl.BlockSpec(shape, lambda b: (0,) * len(shape))
    smem = pl.BlockSpec(memory_space=pltpu.SMEM)
    return pl.pallas_call(
        _mlp2_body,
        grid=(B,),
        in_specs=[smem,
                  pl.BlockSpec((S, TD), lambda b: (b, 0)),
                  pl.BlockSpec((S, 4), lambda b: (b, 0)),
                  full((TD, TD)), full((1, TD)), full((TD, TD)), full((1, TD))],
        out_specs=[pl.BlockSpec((1, S, TD), lambda b: (b, 0, 0)),
                   pl.BlockSpec((1, S, 4), lambda b: (b, 0, 0)),
                   pl.BlockSpec((1, 1, S), lambda b: (b, 0, 0))],
        out_shape=[jax.ShapeDtypeStruct((B, S, TD), jnp.float32),
                   jax.ShapeDtypeStruct((B, S, 4), jnp.float32),
                   jax.ShapeDtypeStruct((B, 1, S), jnp.bool_)],
    )(counts, tok, cen, w0, b0.reshape(1, -1), w1, b1.reshape(1, -1))


def kernel(coords, features, batch_ids, times,
           W1_0, b1_0, W1_1, b1_1, W1_2, b1_2, W1_3, b1_3,
           W2_0, b2_0, W2_1, b2_1):
    bid = batch_ids.astype(jnp.int32)
    counts = jnp.bincount(bid, length=B).astype(jnp.int32)
    offsets = (jnp.cumsum(counts) - counts).astype(jnp.int32)
    p4 = jnp.concatenate([coords[:, :3], times], axis=1)
    xT = p4.T
    bid2 = bid.reshape(1, N)

    pf = _run_mlp1(features, [W1_0, W1_1, W1_2, W1_3], [b1_0, b1_1, b1_2, b1_3])
    cx, cy, cz, ct, gi = _run_fps(xT, p4, bid2)
    cnt16 = jnp.zeros((16,), jnp.int32).at[:B].set(counts)
    off16 = jnp.zeros((16,), jnp.int32).at[:B].set(offsets)
    tok, cenT = _sc_knn_pool(xT, p4, pf, cx, cy, cz, ct, gi, cnt16, off16)
    tokens, centroids, valid3 = _run_mlp2(counts, tok, cenT.T,
                                          W2_0, b2_0, W2_1, b2_1)
    return tokens, centroids, valid3.reshape(B, S)


# speedup vs baseline: 2.5653x; 1.1525x over previous
import functools

import jax
import jax.numpy as jnp
from jax import lax
from jax.experimental import pallas as pl
from jax.experimental.pallas import tpu as pltpu
from jax.experimental.pallas import tpu_sc as plsc

N = 16384
B = 8
S = 128
KNB = 16
FD = 128
TD = 768
INF = 1e10


def _gelu(x):
    return x * 0.5 * (1.0 + jax.lax.erf(x * 0.7071067811865476))


def _bdot(x, w):
    return jnp.dot(x.astype(jnp.bfloat16), w.astype(jnp.bfloat16),
                   preferred_element_type=jnp.float32)


def _mlp1_body(x_ref, w0, b0, w1, b1, w2, b2, w3, b3, o_ref):
    h = _gelu(_bdot(x_ref[...], w0[...]) + b0[...])
    h = _gelu(_bdot(h, w1[...]) + b1[...])
    h = _gelu(_bdot(h, w2[...]) + b2[...])
    o_ref[...] = _bdot(h, w3[...]) + b3[...]


def _run_mlp1(features, ws, bs):
    blk = 2048
    grid = N // blk
    full = lambda shape: pl.BlockSpec(shape, lambda i: (0,) * len(shape))
    in_specs = [pl.BlockSpec((blk, FD), lambda i: (i, 0))]
    for w, b in zip(ws, bs):
        in_specs.append(full(w.shape))
        in_specs.append(full((1,) + b.shape))
    args = [features]
    for w, b in zip(ws, bs):
        args.append(w)
        args.append(b.reshape(1, -1))
    return pl.pallas_call(
        _mlp1_body,
        grid=(grid,),
        in_specs=in_specs,
        out_specs=pl.BlockSpec((blk, TD), lambda i: (i, 0)),
        out_shape=jax.ShapeDtypeStruct((N, TD), jnp.float32),
    )(*args)


def _fps_body(xT_ref, p4_ref, bid_ref, cx_ref, cy_ref, cz_ref, ct_ref, gi_ref):
    xr = xT_ref[0:1, :]
    yr = xT_ref[1:2, :]
    zr = xT_ref[2:3, :]
    tr = xT_ref[3:4, :]
    bid = bid_ref[0:1, :]
    brow = jax.lax.broadcasted_iota(jnp.int32, (B, 1), 0)
    maskB = bid == brow
    gidx = jax.lax.broadcasted_iota(jnp.int32, (B, N), 1)
    lane = jax.lax.broadcasted_iota(jnp.int32, (B, S), 1)
    mind0 = jnp.where(maskB, jnp.float32(INF), jnp.float32(-INF))
    start = jnp.min(jnp.where(maskB, gidx, N), axis=1, keepdims=True)
    zf = jnp.zeros((B, S), jnp.float32)
    zi = jnp.zeros((B, S), jnp.int32)

    def body(s, carry):
        mind, cur, ax, ay, az, at, ai = carry
        onehot = gidx == cur
        cpx = jnp.sum(jnp.where(onehot, xr, 0.0), axis=1, keepdims=True)
        cpy = jnp.sum(jnp.where(onehot, yr, 0.0), axis=1, keepdims=True)
        cpz = jnp.sum(jnp.where(onehot, zr, 0.0), axis=1, keepdims=True)
        cpt = jnp.sum(jnp.where(onehot, tr, 0.0), axis=1, keepdims=True)
        sl = lane == s
        ax = ax + jnp.where(sl, cpx, 0.0)
        ay = ay + jnp.where(sl, cpy, 0.0)
        az = az + jnp.where(sl, cpz, 0.0)
        at = at + jnp.where(sl, cpt, 0.0)
        ai = ai + jnp.where(sl, cur, 0)
        dx = xr - cpx
        d = dx * dx
        dy = yr - cpy
        d = d + dy * dy
        dz = zr - cpz
        d = d + dz * dz
        dt = tr - cpt
        d = d + dt * dt
        mind = jnp.minimum(mind, d)
        m = jnp.max(mind, axis=1, keepdims=True)
        nxt = jnp.min(jnp.where(mind == m, gidx, N), axis=1, keepdims=True)
        return mind, nxt, ax, ay, az, at, ai

    _, _, ax, ay, az, at, ai = jax.lax.fori_loop(
        0, S, body, (mind0, start, zf, zf, zf, zf, zi))
    cx_ref[...] = ax
    cy_ref[...] = ay
    cz_ref[...] = az
    ct_ref[...] = at
    gi_ref[...] = ai


def _run_fps(xT, p4, bid2):
    full = lambda shape: pl.BlockSpec(shape, lambda: (0,) * len(shape))
    outs = [jax.ShapeDtypeStruct((B, S), jnp.float32)] * 4 + [
        jax.ShapeDtypeStruct((B, S), jnp.int32)]
    return pl.pallas_call(
        _fps_body,
        in_specs=[full((4, N)), full((N, 4)), full((1, N))],
        out_specs=[full((B, S))] * 5,
        out_shape=outs,
    )(xT, p4, bid2)


def _knn_body(xT_ref, bid_ref, cx_ref, cy_ref, cz_ref, ct_ref, out_ref):
    b = pl.program_id(0)
    cxb = cx_ref[0, :, :]
    cyb = cy_ref[0, :, :]
    czb = cz_ref[0, :, :]
    ctb = ct_ref[0, :, :]
    xr = xT_ref[0:1, :]
    yr = xT_ref[1:2, :]
    zr = xT_ref[2:3, :]
    tr = xT_ref[3:4, :]
    maskb = bid_ref[0:1, :] == b
    dx = xr - cxb
    d = dx * dx
    dy = yr - cyb
    d = d + dy * dy
    dz = zr - czb
    d = d + dz * dz
    dt = tr - ctb
    d = d + dt * dt
    d = jnp.where(maskb, d, jnp.float32(INF))
    gidx = jax.lax.broadcasted_iota(jnp.int32, (S, N), 1)
    for j in range(KNB):
        m = jnp.min(d, axis=1, keepdims=True)
        ij = jnp.min(jnp.where(d == m, gidx, N), axis=1, keepdims=True)
        out_ref[0, :, j:j + 1] = ij
        d = jnp.where(gidx == ij, jnp.float32(INF), d)


def _run_knn(xT, bid2, cx3, cy3, cz3, ct3):
    full = lambda shape: pl.BlockSpec(shape, lambda b: (0,) * len(shape))
    cspec = pl.BlockSpec((1, S, 1), lambda b: (b, 0, 0))
    return pl.pallas_call(
        _knn_body,
        grid=(B,),
        in_specs=[full((4, N)), full((1, N)), cspec, cspec, cspec, cspec],
        out_specs=pl.BlockSpec((1, S, KNB), lambda b: (b, 0, 0)),
        out_shape=jax.ShapeDtypeStruct((B, S, KNB), jnp.int32),
    )(xT, bid2, cx3, cy3, cz3, ct3)


def _pool_body(pf_ref, knn_ref, cnt_ref, off_ref, tok_ref):
    for b in range(B):
        cnt = cnt_ref[b]
        off = off_ref[b]
        small = cnt <= S

        @pl.when(small)
        def _():
            def inner(s, _):
                idx = jnp.minimum(off + s, N - 1)
                tok_ref[pl.ds(b * S + s, 1), :] = pf_ref[pl.ds(idx, 1), :]
                return 0
            jax.lax.fori_loop(0, S, inner, 0)

        @pl.when(jnp.logical_not(small))
        def _():
            def inner(s, _):
                acc = pf_ref[pl.ds(knn_ref[b, s, 0], 1), :]
                for j in range(1, KNB):
                    acc = jnp.maximum(acc, pf_ref[pl.ds(knn_ref[b, s, j], 1), :])
                tok_ref[pl.ds(b * S + s, 1), :] = acc
                return 0
            jax.lax.fori_loop(0, S, inner, 0)


def _run_pool(pf, knn, counts, offsets):
    full = lambda shape: pl.BlockSpec(shape, lambda: (0,) * len(shape))
    smem = pl.BlockSpec(memory_space=pltpu.SMEM)
    return pl.pallas_call(
        _pool_body,
        in_specs=[full((N, TD)), smem, smem, smem],
        out_specs=full((B * S, TD)),
        out_shape=jax.ShapeDtypeStruct((B * S, TD), jnp.float32),
    )(pf, knn, counts, offsets)


NPAD = N + 16


def _sc_scalar(ref, i):
    v = plsc.load_gather(ref, [jnp.full((16,), i, jnp.int32)])
    return v[0]


def _sc_mesh():
    return plsc.VectorSubcoreMesh(core_axis_name="c", subcore_axis_name="s",
                                  num_cores=2)


def _sc_knn(xT, cx, cy, cz, ct, gi, counts, offsets):
    @functools.partial(
        pl.kernel,
        out_type=[jax.ShapeDtypeStruct((B * S, KNB), jnp.int32),
                  jax.ShapeDtypeStruct((4, B * S), jnp.float32)],
        mesh=_sc_mesh(),
        compiler_params=pltpu.CompilerParams(needs_layout_passes=False),
        scratch_types=[
            pltpu.VMEM((4, NPAD), jnp.float32),
            pltpu.VMEM((32,), jnp.float32),
            pltpu.VMEM((32,), jnp.float32),
            pltpu.VMEM((32,), jnp.float32),
            pltpu.VMEM((32,), jnp.float32),
            pltpu.VMEM((16,), jnp.int32),
            pltpu.VMEM((16,), jnp.int32),
            pltpu.VMEM((32,), jnp.int32),
            pltpu.VMEM((32, KNB), jnp.int32),
            pltpu.VMEM((4, 32), jnp.float32),
        ],
    )
    def body(xT_h, cx_h, cy_h, cz_h, ct_h, gi_h, cnt_h, off_h,
             knn_h, cen_h,
             coords_v, cxv, cyv, czv, ctv, cntv, offv, idx32v, kbuf_v,
             cenr_v):
        cid = lax.axis_index("c")
        sid = lax.axis_index("s")
        wid = sid * 2 + cid
        b = wid // 4
        q = wid % 4
        s0 = q * 32
        lane = lax.broadcasted_iota(jnp.int32, (16,), 0)

        pltpu.sync_copy(cnt_h, cntv)
        pltpu.sync_copy(off_h, offv)
        cnt = _sc_scalar(cntv, b)
        off = _sc_scalar(offv, b)
        small = cnt <= S
        pltpu.sync_copy(xT_h, coords_v.at[:, :N])

        @pl.when(jnp.logical_not(small))
        def _():
            pltpu.sync_copy(gi_h.at[b, pl.ds(s0, 32)], idx32v)
            pltpu.sync_copy(cx_h.at[b, pl.ds(s0, 32)], cxv)
            pltpu.sync_copy(cy_h.at[b, pl.ds(s0, 32)], cyv)
            pltpu.sync_copy(cz_h.at[b, pl.ds(s0, 32)], czv)
            pltpu.sync_copy(ct_h.at[b, pl.ds(s0, 32)], ctv)
            end = off + cnt
            abase = pl.multiple_of((off // 16) * 16, 16)
            nblk = (end - abase + 15) // 16

            def tok_body(sl, _):
                cxs = _sc_scalar(cxv, sl)
                cys = _sc_scalar(cyv, sl)
                czs = _sc_scalar(czv, sl)
                cts = _sc_scalar(ctv, sl)

                def blk(i, carry):
                    tv, ti = carry
                    base = pl.multiple_of(abase + i * 16, 16)
                    dx = coords_v[0, pl.ds(base, 16)] - cxs
                    d = dx * dx
                    dy = coords_v[1, pl.ds(base, 16)] - cys
                    d = d + dy * dy
                    dz = coords_v[2, pl.ds(base, 16)] - czs
                    d = d + dz * dz
                    dt = coords_v[3, pl.ds(base, 16)] - cts
                    d = d + dt * dt
                    iv = base + lane
                    d = jnp.where((iv >= off) & (iv < end), d,
                                  jnp.float32(INF))
                    sd, si = plsc.sort_key_val(d, iv)
                    rd = lax.rev(sd, (0,))
                    ri = lax.rev(si, (0,))
                    keep_new = rd < tv
                    mv = jnp.where(keep_new, rd, tv)
                    mi = jnp.where(keep_new, ri, ti)
                    return tuple(plsc.sort_key_val(mv, mi))

                tv0 = jnp.full((16,), jnp.float32(INF))
                ti0 = jnp.zeros((16,), jnp.int32)
                _, ti = lax.fori_loop(0, nblk, blk, (tv0, ti0))
                kbuf_v[sl, :] = ti
                return 0

            lax.fori_loop(0, 32, tok_body, 0)

        @pl.when(small)
        def _():
            base = off + s0
            idx32v[pl.ds(0, 16)] = jnp.minimum(base + lane, N - 1)
            idx32v[pl.ds(16, 16)] = jnp.minimum(base + 16 + lane, N - 1)

            def tok_body(sl, _):
                row = plsc.load_gather(
                    idx32v, [jnp.full((16,), sl, jnp.int32)])
                kbuf_v[sl, :] = row
                return 0

            lax.fori_loop(0, 32, tok_body, 0)

        t0 = b * S + s0
        for h in (0, 16):
            idx16 = idx32v[pl.ds(h, 16)]
            for c in range(4):
                g = plsc.load_gather(coords_v,
                                     [jnp.full((16,), c, jnp.int32), idx16])
                cenr_v[c, pl.ds(h, 16)] = g
        for c in range(4):
            pltpu.sync_copy(cenr_v.at[c, pl.ds(0, 32)],
                            cen_h.at[c, pl.ds(t0, 32)])
        pltpu.sync_copy(kbuf_v, knn_h.at[pl.ds(t0, 32), :])

    return body(xT, cx, cy, cz, ct, gi, counts, offsets)


def _sc_pool(pf, knn):
    @functools.partial(
        pl.kernel,
        out_type=jax.ShapeDtypeStruct((B * S, TD), jnp.float32),
        mesh=_sc_mesh(),
        compiler_params=pltpu.CompilerParams(needs_layout_passes=False),
        scratch_types=[
            pltpu.VMEM((32, KNB), jnp.int32),
            pltpu.VMEM((16,), jnp.int32),
            pltpu.VMEM((KNB, TD), jnp.float32),
            pltpu.VMEM((32, TD), jnp.float32),
            pltpu.SemaphoreType.DMA,
        ],
    )
    def body(pf_h, knn_h, tok_h, kbuf_v, idxv, rows_v, out_v, sem):
        cid = lax.axis_index("c")
        sid = lax.axis_index("s")
        wid = sid * 2 + cid
        t0 = wid * 32
        pltpu.sync_copy(knn_h.at[pl.ds(t0, 32), :], kbuf_v)

        def tok_body(sl, _):
            idxv[...] = kbuf_v[sl, :]
            pltpu.async_copy(pf_h.at[idxv], rows_v, sem).wait()

            def pool(c, _):
                cb = pl.multiple_of(c * 16, 16)
                acc = rows_v[0, pl.ds(cb, 16)]
                for r in range(1, KNB):
                    acc = jnp.maximum(acc, rows_v[r, pl.ds(cb, 16)])
                out_v[sl, pl.ds(cb, 16)] = acc
                return 0

            lax.fori_loop(0, TD // 16, pool, 0)
            return 0

        lax.fori_loop(0, 32, tok_body, 0)
        pltpu.sync_copy(out_v, tok_h.at[pl.ds(t0, 32), :])

    return body(pf, knn)


def _mlp2_body(cnt_ref, tok_ref, cen_ref, w0, b0, w1, b1,
               tokens_ref, cents_ref, valid_ref):
    b = pl.program_id(0)
    cnt = cnt_ref[b]
    small = cnt <= S
    t = tok_ref[...]
    h = _gelu(_bdot(t, w0[...]) + b0[...])
    h = _bdot(h, w1[...]) + b1[...]
    lim = jnp.where(small, jnp.minimum(cnt, S), S)
    sidx = jax.lax.broadcasted_iota(jnp.int32, (S, 1), 0)
    validc = sidx < lim
    lidx = jax.lax.broadcasted_iota(jnp.int32, (1, S), 1)
    validr = lidx < lim
    tokens_ref[0, :, :] = jnp.where(validc, h, 0.0)
    cents_ref[0, :, :] = jnp.where(validc, cen_ref[...], 0.0)
    valid_ref[0, :, :] = validr


def _run_mlp2(counts, tok, cen, w0, b0, w1, b1):
    full = lambda shape: pl.BlockSpec(shape, lambda b: (0,) * len(shape))
    smem = pl.BlockSpec(memory_space=pltpu.SMEM)
    return pl.pallas_call(
        _mlp2_body,
        grid=(B,),
        in_specs=[smem,
                  pl.BlockSpec((S, TD), lambda b: (b, 0)),
                  pl.BlockSpec((S, 4), lambda b: (b, 0)),
                  full((TD, TD)), full((1, TD)), full((TD, TD)), full((1, TD))],
        out_specs=[pl.BlockSpec((1, S, TD), lambda b: (b, 0, 0)),
                   pl.BlockSpec((1, S, 4), lambda b: (b, 0, 0)),
                   pl.BlockSpec((1, 1, S), lambda b: (b, 0, 0))],
        out_shape=[jax.ShapeDtypeStruct((B, S, TD), jnp.float32),
                   jax.ShapeDtypeStruct((B, S, 4), jnp.float32),
                   jax.ShapeDtypeStruct((B, 1, S), jnp.bool_)],
    )(counts, tok, cen, w0, b0.reshape(1, -1), w1, b1.reshape(1, -1))


def kernel(coords, features, batch_ids, times,
           W1_0, b1_0, W1_1, b1_1, W1_2, b1_2, W1_3, b1_3,
           W2_0, b2_0, W2_1, b2_1):
    bid = batch_ids.astype(jnp.int32)
    counts = jnp.bincount(bid, length=B).astype(jnp.int32)
    offsets = (jnp.cumsum(counts) - counts).astype(jnp.int32)
    p4 = jnp.concatenate([coords[:, :3], times], axis=1)
    xT = p4.T
    bid2 = bid.reshape(1, N)

    pf = _run_mlp1(features, [W1_0, W1_1, W1_2, W1_3], [b1_0, b1_1, b1_2, b1_3])
    cx, cy, cz, ct, gi = _run_fps(xT, p4, bid2)
    cnt16 = jnp.zeros((16,), jnp.int32).at[:B].set(counts)
    off16 = jnp.zeros((16,), jnp.int32).at[:B].set(offsets)
    knn, cenT = _sc_knn(xT, cx, cy, cz, ct, gi, cnt16, off16)
    tok = _sc_pool(pf, knn)
    tokens, centroids, valid3 = _run_mlp2(counts, tok, cenT.T,
                                          W2_0, b2_0, W2_1, b2_1)
    return tokens, centroids, valid3.reshape(B, S)


# speedup vs baseline: 2.5844x; 1.0074x over previous
import functools

import jax
import jax.numpy as jnp
from jax import lax
from jax.experimental import pallas as pl
from jax.experimental.pallas import tpu as pltpu
from jax.experimental.pallas import tpu_sc as plsc

N = 16384
B = 8
S = 128
KNB = 16
FD = 128
TD = 768
INF = 1e10


def _gelu(x):
    return x * 0.5 * (1.0 + jax.lax.erf(x * 0.7071067811865476))


def _bdot(x, w):
    return jnp.dot(x.astype(jnp.bfloat16), w.astype(jnp.bfloat16),
                   preferred_element_type=jnp.float32)


def _mlp1_body(x_ref, w0, b0, w1, b1, w2, b2, w3, b3, o_ref):
    h = _gelu(_bdot(x_ref[...], w0[...]) + b0[...])
    h = _gelu(_bdot(h, w1[...]) + b1[...])
    h = _gelu(_bdot(h, w2[...]) + b2[...])
    o_ref[...] = _bdot(h, w3[...]) + b3[...]


def _run_mlp1(features, ws, bs):
    blk = 2048
    grid = N // blk
    full = lambda shape: pl.BlockSpec(shape, lambda i: (0,) * len(shape))
    in_specs = [pl.BlockSpec((blk, FD), lambda i: (i, 0))]
    for w, b in zip(ws, bs):
        in_specs.append(full(w.shape))
        in_specs.append(full((1,) + b.shape))
    args = [features]
    for w, b in zip(ws, bs):
        args.append(w)
        args.append(b.reshape(1, -1))
    return pl.pallas_call(
        _mlp1_body,
        grid=(grid,),
        in_specs=in_specs,
        out_specs=pl.BlockSpec((blk, TD), lambda i: (i, 0)),
        out_shape=jax.ShapeDtypeStruct((N, TD), jnp.float32),
    )(*args)


def _fps_body(xT_ref, p4_ref, bid_ref, cx_ref, cy_ref, cz_ref, ct_ref, gi_ref):
    xr = xT_ref[0:1, :]
    yr = xT_ref[1:2, :]
    zr = xT_ref[2:3, :]
    tr = xT_ref[3:4, :]
    bid = bid_ref[0:1, :]
    brow = jax.lax.broadcasted_iota(jnp.int32, (B, 1), 0)
    maskB = bid == brow
    gidx = jax.lax.broadcasted_iota(jnp.int32, (B, N), 1)
    lane = jax.lax.broadcasted_iota(jnp.int32, (B, S), 1)
    mind0 = jnp.where(maskB, jnp.float32(INF), jnp.float32(-INF))
    start = jnp.min(jnp.where(maskB, gidx, N), axis=1, keepdims=True)
    zf = jnp.zeros((B, S), jnp.float32)
    zi = jnp.zeros((B, S), jnp.int32)

    def body(s, carry):
        mind, cur, ax, ay, az, at, ai = carry
        onehot = gidx == cur
        cpx = jnp.sum(jnp.where(onehot, xr, 0.0), axis=1, keepdims=True)
        cpy = jnp.sum(jnp.where(onehot, yr, 0.0), axis=1, keepdims=True)
        cpz = jnp.sum(jnp.where(onehot, zr, 0.0), axis=1, keepdims=True)
        cpt = jnp.sum(jnp.where(onehot, tr, 0.0), axis=1, keepdims=True)
        sl = lane == s
        ax = ax + jnp.where(sl, cpx, 0.0)
        ay = ay + jnp.where(sl, cpy, 0.0)
        az = az + jnp.where(sl, cpz, 0.0)
        at = at + jnp.where(sl, cpt, 0.0)
        ai = ai + jnp.where(sl, cur, 0)
        dx = xr - cpx
        d = dx * dx
        dy = yr - cpy
        d = d + dy * dy
        dz = zr - cpz
        d = d + dz * dz
        dt = tr - cpt
        d = d + dt * dt
        mind = jnp.minimum(mind, d)
        m = jnp.max(mind, axis=1, keepdims=True)
        nxt = jnp.min(jnp.where(mind == m, gidx, N), axis=1, keepdims=True)
        return mind, nxt, ax, ay, az, at, ai

    _, _, ax, ay, az, at, ai = jax.lax.fori_loop(
        0, S, body, (mind0, start, zf, zf, zf, zf, zi))
    cx_ref[...] = ax
    cy_ref[...] = ay
    cz_ref[...] = az
    ct_ref[...] = at
    gi_ref[...] = ai


def _run_fps(xT, p4, bid2):
    full = lambda shape: pl.BlockSpec(shape, lambda: (0,) * len(shape))
    outs = [jax.ShapeDtypeStruct((B, S), jnp.float32)] * 4 + [
        jax.ShapeDtypeStruct((B, S), jnp.int32)]
    return pl.pallas_call(
        _fps_body,
        in_specs=[full((4, N)), full((N, 4)), full((1, N))],
        out_specs=[full((B, S))] * 5,
        out_shape=outs,
    )(xT, p4, bid2)


def _knn_body(xT_ref, bid_ref, cx_ref, cy_ref, cz_ref, ct_ref, out_ref):
    b = pl.program_id(0)
    cxb = cx_ref[0, :, :]
    cyb = cy_ref[0, :, :]
    czb = cz_ref[0, :, :]
    ctb = ct_ref[0, :, :]
    xr = xT_ref[0:1, :]
    yr = xT_ref[1:2, :]
    zr = xT_ref[2:3, :]
    tr = xT_ref[3:4, :]
    maskb = bid_ref[0:1, :] == b
    dx = xr - cxb
    d = dx * dx
    dy = yr - cyb
    d = d + dy * dy
    dz = zr - czb
    d = d + dz * dz
    dt = tr - ctb
    d = d + dt * dt
    d = jnp.where(maskb, d, jnp.float32(INF))
    gidx = jax.lax.broadcasted_iota(jnp.int32, (S, N), 1)
    for j in range(KNB):
        m = jnp.min(d, axis=1, keepdims=True)
        ij = jnp.min(jnp.where(d == m, gidx, N), axis=1, keepdims=True)
        out_ref[0, :, j:j + 1] = ij
        d = jnp.where(gidx == ij, jnp.float32(INF), d)


def _run_knn(xT, bid2, cx3, cy3, cz3, ct3):
    full = lambda shape: pl.BlockSpec(shape, lambda b: (0,) * len(shape))
    cspec = pl.BlockSpec((1, S, 1), lambda b: (b, 0, 0))
    return pl.pallas_call(
        _knn_body,
        grid=(B,),
        in_specs=[full((4, N)), full((1, N)), cspec, cspec, cspec, cspec],
        out_specs=pl.BlockSpec((1, S, KNB), lambda b: (b, 0, 0)),
        out_shape=jax.ShapeDtypeStruct((B, S, KNB), jnp.int32),
    )(xT, bid2, cx3, cy3, cz3, ct3)


def _pool_body(pf_ref, knn_ref, cnt_ref, off_ref, tok_ref):
    for b in range(B):
        cnt = cnt_ref[b]
        off = off_ref[b]
        small = cnt <= S

        @pl.when(small)
        def _():
            def inner(s, _):
                idx = jnp.minimum(off + s, N - 1)
                tok_ref[pl.ds(b * S + s, 1), :] = pf_ref[pl.ds(idx, 1), :]
                return 0
            jax.lax.fori_loop(0, S, inner, 0)

        @pl.when(jnp.logical_not(small))
        def _():
            def inner(s, _):
                acc = pf_ref[pl.ds(knn_ref[b, s, 0], 1), :]
                for j in range(1, KNB):
                    acc = jnp.maximum(acc, pf_ref[pl.ds(knn_ref[b, s, j], 1), :])
                tok_ref[pl.ds(b * S + s, 1), :] = acc
                return 0
            jax.lax.fori_loop(0, S, inner, 0)


def _run_pool(pf, knn, counts, offsets):
    full = lambda shape: pl.BlockSpec(shape, lambda: (0,) * len(shape))
    smem = pl.BlockSpec(memory_space=pltpu.SMEM)
    return pl.pallas_call(
        _pool_body,
        in_specs=[full((N, TD)), smem, smem, smem],
        out_specs=full((B * S, TD)),
        out_shape=jax.ShapeDtypeStruct((B * S, TD), jnp.float32),
    )(pf, knn, counts, offsets)


NPAD = N + 16


def _sc_scalar(ref, i):
    v = plsc.load_gather(ref, [jnp.full((16,), i, jnp.int32)])
    return v[0]


def _sc_mesh():
    return plsc.VectorSubcoreMesh(core_axis_name="c", subcore_axis_name="s",
                                  num_cores=2)


def _sc_knn(xT, cx, cy, cz, ct, gi, counts, offsets):
    @functools.partial(
        pl.kernel,
        out_type=[jax.ShapeDtypeStruct((B * S, KNB), jnp.int32),
                  jax.ShapeDtypeStruct((4, B * S), jnp.float32)],
        mesh=_sc_mesh(),
        compiler_params=pltpu.CompilerParams(needs_layout_passes=False),
        scratch_types=[
            pltpu.VMEM((4, NPAD), jnp.float32),
            pltpu.VMEM((32,), jnp.float32),
            pltpu.VMEM((32,), jnp.float32),
            pltpu.VMEM((32,), jnp.float32),
            pltpu.VMEM((32,), jnp.float32),
            pltpu.VMEM((16,), jnp.int32),
            pltpu.VMEM((16,), jnp.int32),
            pltpu.VMEM((32,), jnp.int32),
            pltpu.VMEM((32, KNB), jnp.int32),
            pltpu.VMEM((4, 32), jnp.float32),
        ],
    )
    def body(xT_h, cx_h, cy_h, cz_h, ct_h, gi_h, cnt_h, off_h,
             knn_h, cen_h,
             coords_v, cxv, cyv, czv, ctv, cntv, offv, idx32v, kbuf_v,
             cenr_v):
        cid = lax.axis_index("c")
        sid = lax.axis_index("s")
        wid = sid * 2 + cid
        b = wid // 4
        q = wid % 4
        s0 = q * 32
        lane = lax.broadcasted_iota(jnp.int32, (16,), 0)

        pltpu.sync_copy(cnt_h, cntv)
        pltpu.sync_copy(off_h, offv)
        cnt = _sc_scalar(cntv, b)
        off = _sc_scalar(offv, b)
        small = cnt <= S
        pltpu.sync_copy(xT_h, coords_v.at[:, :N])

        @pl.when(jnp.logical_not(small))
        def _():
            pltpu.sync_copy(gi_h.at[b, pl.ds(s0, 32)], idx32v)
            pltpu.sync_copy(cx_h.at[b, pl.ds(s0, 32)], cxv)
            pltpu.sync_copy(cy_h.at[b, pl.ds(s0, 32)], cyv)
            pltpu.sync_copy(cz_h.at[b, pl.ds(s0, 32)], czv)
            pltpu.sync_copy(ct_h.at[b, pl.ds(s0, 32)], ctv)
            end = off + cnt
            abase = pl.multiple_of((off // 16) * 16, 16)
            nblk = (end - abase + 15) // 16

            def tok_body(sl, _):
                cxs = _sc_scalar(cxv, sl)
                cys = _sc_scalar(cyv, sl)
                czs = _sc_scalar(czv, sl)
                cts = _sc_scalar(ctv, sl)

                def blk(i, carry):
                    tv, ti = carry
                    base = pl.multiple_of(abase + i * 16, 16)
                    dx = coords_v[0, pl.ds(base, 16)] - cxs
                    d = dx * dx
                    dy = coords_v[1, pl.ds(base, 16)] - cys
                    d = d + dy * dy
                    dz = coords_v[2, pl.ds(base, 16)] - czs
                    d = d + dz * dz
                    dt = coords_v[3, pl.ds(base, 16)] - cts
                    d = d + dt * dt
                    iv = base + lane
                    d = jnp.where((iv >= off) & (iv < end), d,
                                  jnp.float32(INF))
                    sd, si = plsc.sort_key_val(d, iv)
                    rd = lax.rev(sd, (0,))
                    ri = lax.rev(si, (0,))
                    keep_new = rd < tv
                    mv = jnp.where(keep_new, rd, tv)
                    mi = jnp.where(keep_new, ri, ti)
                    return tuple(plsc.sort_key_val(mv, mi))

                tv0 = jnp.full((16,), jnp.float32(INF))
                ti0 = jnp.zeros((16,), jnp.int32)
                _, ti = lax.fori_loop(0, nblk, blk, (tv0, ti0))
                kbuf_v[sl, :] = ti
                return 0

            lax.fori_loop(0, 32, tok_body, 0)

        @pl.when(small)
        def _():
            base = off + s0
            idx32v[pl.ds(0, 16)] = jnp.minimum(base + lane, N - 1)
            idx32v[pl.ds(16, 16)] = jnp.minimum(base + 16 + lane, N - 1)

            def tok_body(sl, _):
                row = plsc.load_gather(
                    idx32v, [jnp.full((16,), sl, jnp.int32)])
                kbuf_v[sl, :] = row
                return 0

            lax.fori_loop(0, 32, tok_body, 0)

        t0 = b * S + s0
        for h in (0, 16):
            idx16 = idx32v[pl.ds(h, 16)]
            for c in range(4):
                g = plsc.load_gather(coords_v,
                                     [jnp.full((16,), c, jnp.int32), idx16])
                cenr_v[c, pl.ds(h, 16)] = g
        for c in range(4):
            pltpu.sync_copy(cenr_v.at[c, pl.ds(0, 32)],
                            cen_h.at[c, pl.ds(t0, 32)])
        pltpu.sync_copy(kbuf_v, knn_h.at[pl.ds(t0, 32), :])

    return body(xT, cx, cy, cz, ct, gi, counts, offsets)


def _sc_pool(pf, knn):
    @functools.partial(
        pl.kernel,
        out_type=jax.ShapeDtypeStruct((B * S, TD), jnp.float32),
        mesh=_sc_mesh(),
        compiler_params=pltpu.CompilerParams(needs_layout_passes=False),
        scratch_types=[
            pltpu.VMEM((32, KNB), jnp.int32),
            pltpu.VMEM((16,), jnp.int32),
            pltpu.VMEM((KNB, TD), jnp.float32),
            pltpu.VMEM((32, TD), jnp.float32),
            pltpu.SemaphoreType.DMA,
        ],
    )
    def body(pf_h, knn_h, tok_h, kbuf_v, idxv, rows_v, out_v, sem):
        cid = lax.axis_index("c")
        sid = lax.axis_index("s")
        wid = sid * 2 + cid
        t0 = wid * 32
        pltpu.sync_copy(knn_h.at[pl.ds(t0, 32), :], kbuf_v)

        def tok_body(sl, _):
            idxv[...] = kbuf_v[sl, :]
            pltpu.async_copy(pf_h.at[idxv], rows_v, sem).wait()

            def pool(c, _):
                cb = pl.multiple_of(c * 16, 16)
                acc = rows_v[0, pl.ds(cb, 16)]
                for r in range(1, KNB):
                    acc = jnp.maximum(acc, rows_v[r, pl.ds(cb, 16)])
                out_v[sl, pl.ds(cb, 16)] = acc
                return 0

            lax.fori_loop(0, TD // 16, pool, 0)
            return 0

        lax.fori_loop(0, 32, tok_body, 0)
        pltpu.sync_copy(out_v, tok_h.at[pl.ds(t0, 32), :])

    return body(pf, knn)


def _mlp2_body(cnt_ref, tok_ref, cen_ref, w0, b0, w1, b1,
               tokens_ref, cents_ref, valid_ref):
    t = tok_ref[...]
    h = _gelu(_bdot(t, w0[...]) + b0[...])
    h = _bdot(h, w1[...]) + b1[...]
    sidx = jax.lax.broadcasted_iota(jnp.int32, (B * S, 1), 0)
    lidx = jax.lax.broadcasted_iota(jnp.int32, (1, B * S), 1)
    limc = jnp.zeros((B * S, 1), jnp.int32)
    limr = jnp.zeros((1, B * S), jnp.int32)
    for b in range(B):
        cnt = cnt_ref[b]
        lim = jnp.where(cnt <= S, jnp.minimum(cnt, S), S) + b * S
        inb_c = (sidx >= b * S) & (sidx < (b + 1) * S)
        limc = jnp.where(inb_c, lim, limc)
        inb_r = (lidx >= b * S) & (lidx < (b + 1) * S)
        limr = jnp.where(inb_r, lim, limr)
    validc = sidx < limc
    tokens_ref[...] = jnp.where(validc, h, 0.0)
    cents_ref[...] = jnp.where(validc, cen_ref[...], 0.0)
    valid_ref[...] = lidx < limr


def _run_mlp2(counts, tok, cen, w0, b0, w1, b1):
    full = lambda shape: pl.BlockSpec(shape, lambda: (0,) * len(shape))
    smem = pl.BlockSpec(memory_space=pltpu.SMEM)
    return pl.pallas_call(
        _mlp2_body,
        in_specs=[smem, full((B * S, TD)), full((B * S, 4)),
                  full((TD, TD)), full((1, TD)), full((TD, TD)), full((1, TD))],
        out_specs=[full((B * S, TD)), full((B * S, 4)), full((1, B * S))],
        out_shape=[jax.ShapeDtypeStruct((B * S, TD), jnp.float32),
                   jax.ShapeDtypeStruct((B * S, 4), jnp.float32),
                   jax.ShapeDtypeStruct((1, B * S), jnp.bool_)],
    )(counts, tok, cen, w0, b0.reshape(1, -1), w1, b1.reshape(1, -1))


def kernel(coords, features, batch_ids, times,
           W1_0, b1_0, W1_1, b1_1, W1_2, b1_2, W1_3, b1_3,
           W2_0, b2_0, W2_1, b2_1):
    bid = batch_ids.astype(jnp.int32)
    counts = jnp.bincount(bid, length=B).astype(jnp.int32)
    offsets = (jnp.cumsum(counts) - counts).astype(jnp.int32)
    p4 = jnp.concatenate([coords[:, :3], times], axis=1)
    xT = p4.T
    bid2 = bid.reshape(1, N)

    pf = _run_mlp1(features, [W1_0, W1_1, W1_2, W1_3], [b1_0, b1_1, b1_2, b1_3])
    cx, cy, cz, ct, gi = _run_fps(xT, p4, bid2)
    cnt16 = jnp.zeros((16,), jnp.int32).at[:B].set(counts)
    off16 = jnp.zeros((16,), jnp.int32).at[:B].set(offsets)
    knn, cenT = _sc_knn(xT, cx, cy, cz, ct, gi, cnt16, off16)
    tok = _sc_pool(pf, knn)
    tokens, centroids, valid2 = _run_mlp2(counts, tok, cenT.T,
                                          W2_0, b2_0, W2_1, b2_1)
    return (tokens.reshape(B, S, TD), centroids.reshape(B, S, 4),
            valid2.reshape(B, S))


# speedup vs baseline: 2.8894x; 1.1180x over previous
import functools

import jax
import jax.numpy as jnp
from jax import lax
from jax.experimental import pallas as pl
from jax.experimental.pallas import tpu as pltpu
from jax.experimental.pallas import tpu_sc as plsc

N = 16384
B = 8
S = 128
KNB = 16
FD = 128
TD = 768
INF = 1e10


def _gelu(x):
    return x * 0.5 * (1.0 + jax.lax.erf(x * 0.7071067811865476))


def _bdot(x, w):
    return jnp.dot(x.astype(jnp.bfloat16), w.astype(jnp.bfloat16),
                   preferred_element_type=jnp.float32)


def _mlp1_body(x_ref, w0, b0, w1, b1, w2, b2, w3, b3, o_ref):
    h = _gelu(_bdot(x_ref[...], w0[...]) + b0[...])
    h = _gelu(_bdot(h, w1[...]) + b1[...])
    h = _gelu(_bdot(h, w2[...]) + b2[...])
    o_ref[...] = _bdot(h, w3[...]) + b3[...]


def _run_mlp1(features, ws, bs):
    blk = 2048
    grid = N // blk
    full = lambda shape: pl.BlockSpec(shape, lambda i: (0,) * len(shape))
    in_specs = [pl.BlockSpec((blk, FD), lambda i: (i, 0))]
    for w, b in zip(ws, bs):
        in_specs.append(full(w.shape))
        in_specs.append(full((1,) + b.shape))
    args = [features]
    for w, b in zip(ws, bs):
        args.append(w)
        args.append(b.reshape(1, -1))
    return pl.pallas_call(
        _mlp1_body,
        grid=(grid,),
        in_specs=in_specs,
        out_specs=pl.BlockSpec((blk, TD), lambda i: (i, 0)),
        out_shape=jax.ShapeDtypeStruct((N, TD), jnp.float32),
    )(*args)


def _fps_body(xT_ref, p4_ref, bid_ref, cx_ref, cy_ref, cz_ref, ct_ref, gi_ref):
    xr = xT_ref[0:1, :]
    yr = xT_ref[1:2, :]
    zr = xT_ref[2:3, :]
    tr = xT_ref[3:4, :]
    bid = bid_ref[0:1, :]
    brow = jax.lax.broadcasted_iota(jnp.int32, (B, 1), 0)
    maskB = bid == brow
    gidx = jax.lax.broadcasted_iota(jnp.int32, (B, N), 1)
    lane = jax.lax.broadcasted_iota(jnp.int32, (B, S), 1)
    mind0 = jnp.where(maskB, jnp.float32(INF), jnp.float32(-INF))
    start = jnp.min(jnp.where(maskB, gidx, N), axis=1, keepdims=True)
    zf = jnp.zeros((B, S), jnp.float32)
    zi = jnp.zeros((B, S), jnp.int32)

    def body(s, carry):
        mind, cur, ax, ay, az, at, ai = carry
        onehot = gidx == cur
        cpx = jnp.sum(jnp.where(onehot, xr, 0.0), axis=1, keepdims=True)
        cpy = jnp.sum(jnp.where(onehot, yr, 0.0), axis=1, keepdims=True)
        cpz = jnp.sum(jnp.where(onehot, zr, 0.0), axis=1, keepdims=True)
        cpt = jnp.sum(jnp.where(onehot, tr, 0.0), axis=1, keepdims=True)
        sl = lane == s
        ax = ax + jnp.where(sl, cpx, 0.0)
        ay = ay + jnp.where(sl, cpy, 0.0)
        az = az + jnp.where(sl, cpz, 0.0)
        at = at + jnp.where(sl, cpt, 0.0)
        ai = ai + jnp.where(sl, cur, 0)
        dx = xr - cpx
        d = dx * dx
        dy = yr - cpy
        d = d + dy * dy
        dz = zr - cpz
        d = d + dz * dz
        dt = tr - cpt
        d = d + dt * dt
        mind = jnp.minimum(mind, d)
        m = jnp.max(mind, axis=1, keepdims=True)
        nxt = jnp.min(jnp.where(mind == m, gidx, N), axis=1, keepdims=True)
        return mind, nxt, ax, ay, az, at, ai

    _, _, ax, ay, az, at, ai = jax.lax.fori_loop(
        0, S, body, (mind0, start, zf, zf, zf, zf, zi))
    cx_ref[...] = ax
    cy_ref[...] = ay
    cz_ref[...] = az
    ct_ref[...] = at
    gi_ref[...] = ai


def _run_fps(xT, p4, bid2):
    full = lambda shape: pl.BlockSpec(shape, lambda: (0,) * len(shape))
    outs = [jax.ShapeDtypeStruct((B, S), jnp.float32)] * 4 + [
        jax.ShapeDtypeStruct((B, S), jnp.int32)]
    return pl.pallas_call(
        _fps_body,
        in_specs=[full((4, N)), full((N, 4)), full((1, N))],
        out_specs=[full((B, S))] * 5,
        out_shape=outs,
    )(xT, p4, bid2)


def _knn_body(xT_ref, bid_ref, cx_ref, cy_ref, cz_ref, ct_ref, out_ref):
    b = pl.program_id(0)
    cxb = cx_ref[0, :, :]
    cyb = cy_ref[0, :, :]
    czb = cz_ref[0, :, :]
    ctb = ct_ref[0, :, :]
    xr = xT_ref[0:1, :]
    yr = xT_ref[1:2, :]
    zr = xT_ref[2:3, :]
    tr = xT_ref[3:4, :]
    maskb = bid_ref[0:1, :] == b
    dx = xr - cxb
    d = dx * dx
    dy = yr - cyb
    d = d + dy * dy
    dz = zr - czb
    d = d + dz * dz
    dt = tr - ctb
    d = d + dt * dt
    d = jnp.where(maskb, d, jnp.float32(INF))
    gidx = jax.lax.broadcasted_iota(jnp.int32, (S, N), 1)
    for j in range(KNB):
        m = jnp.min(d, axis=1, keepdims=True)
        ij = jnp.min(jnp.where(d == m, gidx, N), axis=1, keepdims=True)
        out_ref[0, :, j:j + 1] = ij
        d = jnp.where(gidx == ij, jnp.float32(INF), d)


def _run_knn(xT, bid2, cx3, cy3, cz3, ct3):
    full = lambda shape: pl.BlockSpec(shape, lambda b: (0,) * len(shape))
    cspec = pl.BlockSpec((1, S, 1), lambda b: (b, 0, 0))
    return pl.pallas_call(
        _knn_body,
        grid=(B,),
        in_specs=[full((4, N)), full((1, N)), cspec, cspec, cspec, cspec],
        out_specs=pl.BlockSpec((1, S, KNB), lambda b: (b, 0, 0)),
        out_shape=jax.ShapeDtypeStruct((B, S, KNB), jnp.int32),
    )(xT, bid2, cx3, cy3, cz3, ct3)


def _pool_body(pf_ref, knn_ref, cnt_ref, off_ref, tok_ref):
    for b in range(B):
        cnt = cnt_ref[b]
        off = off_ref[b]
        small = cnt <= S

        @pl.when(small)
        def _():
            def inner(s, _):
                idx = jnp.minimum(off + s, N - 1)
                tok_ref[pl.ds(b * S + s, 1), :] = pf_ref[pl.ds(idx, 1), :]
                return 0
            jax.lax.fori_loop(0, S, inner, 0)

        @pl.when(jnp.logical_not(small))
        def _():
            def inner(s, _):
                acc = pf_ref[pl.ds(knn_ref[b, s, 0], 1), :]
                for j in range(1, KNB):
                    acc = jnp.maximum(acc, pf_ref[pl.ds(knn_ref[b, s, j], 1), :])
                tok_ref[pl.ds(b * S + s, 1), :] = acc
                return 0
            jax.lax.fori_loop(0, S, inner, 0)


def _run_pool(pf, knn, counts, offsets):
    full = lambda shape: pl.BlockSpec(shape, lambda: (0,) * len(shape))
    smem = pl.BlockSpec(memory_space=pltpu.SMEM)
    return pl.pallas_call(
        _pool_body,
        in_specs=[full((N, TD)), smem, smem, smem],
        out_specs=full((B * S, TD)),
        out_shape=jax.ShapeDtypeStruct((B * S, TD), jnp.float32),
    )(pf, knn, counts, offsets)


NPAD = N + 16


def _sc_scalar(ref, i):
    v = plsc.load_gather(ref, [jnp.full((16,), i, jnp.int32)])
    return v[0]


def _sc_mesh():
    return plsc.VectorSubcoreMesh(core_axis_name="c", subcore_axis_name="s",
                                  num_cores=2)


def _sc_knn(xT, cx, cy, cz, ct, gi, counts, offsets):
    @functools.partial(
        pl.kernel,
        out_type=[jax.ShapeDtypeStruct((B * S, KNB), jnp.int32),
                  jax.ShapeDtypeStruct((4, B * S), jnp.float32)],
        mesh=_sc_mesh(),
        compiler_params=pltpu.CompilerParams(needs_layout_passes=False),
        scratch_types=[
            pltpu.VMEM((4, NPAD), jnp.float32),
            pltpu.VMEM((32,), jnp.float32),
            pltpu.VMEM((32,), jnp.float32),
            pltpu.VMEM((32,), jnp.float32),
            pltpu.VMEM((32,), jnp.float32),
            pltpu.VMEM((16,), jnp.int32),
            pltpu.VMEM((16,), jnp.int32),
            pltpu.VMEM((32,), jnp.int32),
            pltpu.VMEM((32, KNB), jnp.int32),
            pltpu.VMEM((4, 32), jnp.float32),
        ],
    )
    def body(xT_h, cx_h, cy_h, cz_h, ct_h, gi_h, cnt_h, off_h,
             knn_h, cen_h,
             coords_v, cxv, cyv, czv, ctv, cntv, offv, idx32v, kbuf_v,
             cenr_v):
        cid = lax.axis_index("c")
        sid = lax.axis_index("s")
        wid = sid * 2 + cid
        b = wid // 4
        q = wid % 4
        s0 = q * 32
        lane = lax.broadcasted_iota(jnp.int32, (16,), 0)

        pltpu.sync_copy(cnt_h, cntv)
        pltpu.sync_copy(off_h, offv)
        cnt = _sc_scalar(cntv, b)
        off = _sc_scalar(offv, b)
        small = cnt <= S
        pltpu.sync_copy(xT_h, coords_v.at[:, :N])

        @pl.when(jnp.logical_not(small))
        def _():
            pltpu.sync_copy(gi_h.at[b, pl.ds(s0, 32)], idx32v)
            pltpu.sync_copy(cx_h.at[b, pl.ds(s0, 32)], cxv)
            pltpu.sync_copy(cy_h.at[b, pl.ds(s0, 32)], cyv)
            pltpu.sync_copy(cz_h.at[b, pl.ds(s0, 32)], czv)
            pltpu.sync_copy(ct_h.at[b, pl.ds(s0, 32)], ctv)
            end = off + cnt
            abase = pl.multiple_of((off // 16) * 16, 16)
            nblk = (end - abase + 15) // 16

            def tok_body(sl, _):
                cxs = _sc_scalar(cxv, sl)
                cys = _sc_scalar(cyv, sl)
                czs = _sc_scalar(czv, sl)
                cts = _sc_scalar(ctv, sl)

                def blk(i, carry):
                    tv, ti = carry
                    base = pl.multiple_of(abase + i * 16, 16)
                    dx = coords_v[0, pl.ds(base, 16)] - cxs
                    d = dx * dx
                    dy = coords_v[1, pl.ds(base, 16)] - cys
                    d = d + dy * dy
                    dz = coords_v[2, pl.ds(base, 16)] - czs
                    d = d + dz * dz
                    dt = coords_v[3, pl.ds(base, 16)] - cts
                    d = d + dt * dt
                    iv = base + lane
                    d = jnp.where((iv >= off) & (iv < end), d,
                                  jnp.float32(INF))
                    sd, si = plsc.sort_key_val(d, iv)
                    rd = lax.rev(sd, (0,))
                    ri = lax.rev(si, (0,))
                    keep_new = rd < tv
                    mv = jnp.where(keep_new, rd, tv)
                    mi = jnp.where(keep_new, ri, ti)
                    return tuple(plsc.sort_key_val(mv, mi))

                tv0 = jnp.full((16,), jnp.float32(INF))
                ti0 = jnp.zeros((16,), jnp.int32)
                _, ti = lax.fori_loop(0, nblk, blk, (tv0, ti0))
                kbuf_v[sl, :] = ti
                return 0

            lax.fori_loop(0, 32, tok_body, 0)

        @pl.when(small)
        def _():
            base = off + s0
            idx32v[pl.ds(0, 16)] = jnp.minimum(base + lane, N - 1)
            idx32v[pl.ds(16, 16)] = jnp.minimum(base + 16 + lane, N - 1)

            def tok_body(sl, _):
                row = plsc.load_gather(
                    idx32v, [jnp.full((16,), sl, jnp.int32)])
                kbuf_v[sl, :] = row
                return 0

            lax.fori_loop(0, 32, tok_body, 0)

        t0 = b * S + s0
        for h in (0, 16):
            idx16 = idx32v[pl.ds(h, 16)]
            for c in range(4):
                g = plsc.load_gather(coords_v,
                                     [jnp.full((16,), c, jnp.int32), idx16])
                cenr_v[c, pl.ds(h, 16)] = g
        for c in range(4):
            pltpu.sync_copy(cenr_v.at[c, pl.ds(0, 32)],
                            cen_h.at[c, pl.ds(t0, 32)])
        pltpu.sync_copy(kbuf_v, knn_h.at[pl.ds(t0, 32), :])

    return body(xT, cx, cy, cz, ct, gi, counts, offsets)


def _sc_pool(pf, knn):
    @functools.partial(
        pl.kernel,
        out_type=jax.ShapeDtypeStruct((B * S, TD), jnp.float32),
        mesh=_sc_mesh(),
        compiler_params=pltpu.CompilerParams(needs_layout_passes=False),
        scratch_types=[
            pltpu.VMEM((32, KNB), jnp.int32),
            pltpu.VMEM((16,), jnp.int32),
            pltpu.VMEM((16,), jnp.int32),
            pltpu.VMEM((KNB, TD), jnp.float32),
            pltpu.VMEM((KNB, TD), jnp.float32),
            pltpu.VMEM((32, TD), jnp.float32),
            pltpu.SemaphoreType.DMA,
            pltpu.SemaphoreType.DMA,
        ],
    )
    def body(pf_h, knn_h, tok_h, kbuf_v, idx0, idx1, rows0, rows1, out_v,
             sem0, sem1):
        cid = lax.axis_index("c")
        sid = lax.axis_index("s")
        wid = sid * 2 + cid
        t0 = wid * 32
        pltpu.sync_copy(knn_h.at[pl.ds(t0, 32), :], kbuf_v)

        def pool(rows_v, sl):
            def step(c, _):
                cb = pl.multiple_of(c * 16, 16)
                acc = rows_v[0, pl.ds(cb, 16)]
                for r in range(1, KNB):
                    acc = jnp.maximum(acc, rows_v[r, pl.ds(cb, 16)])
                out_v[sl, pl.ds(cb, 16)] = acc
                return 0
            lax.fori_loop(0, TD // 16, step, 0)

        idx0[...] = kbuf_v[0, :]
        pltpu.async_copy(pf_h.at[idx0], rows0, sem0)

        def pair(p, _):
            ta = 2 * p
            idx1[...] = kbuf_v[ta + 1, :]
            pltpu.async_copy(pf_h.at[idx1], rows1, sem1)
            pltpu.make_async_copy(pf_h.at[idx0], rows0, sem0).wait()
            pool(rows0, ta)

            @pl.when(p < 15)
            def _():
                idx0[...] = kbuf_v[ta + 2, :]
                pltpu.async_copy(pf_h.at[idx0], rows0, sem0)

            pltpu.make_async_copy(pf_h.at[idx1], rows1, sem1).wait()
            pool(rows1, ta + 1)
            return 0

        lax.fori_loop(0, 16, pair, 0)
        pltpu.sync_copy(out_v, tok_h.at[pl.ds(t0, 32), :])

    return body(pf, knn)


def _mlp2_body(cnt_ref, tok_ref, cen_ref, w0, b0, w1, b1,
               tokens_ref, cents_ref, valid_ref):
    t = tok_ref[...]
    h = _gelu(_bdot(t, w0[...]) + b0[...])
    h = _bdot(h, w1[...]) + b1[...]
    sidx = jax.lax.broadcasted_iota(jnp.int32, (B * S, 1), 0)
    lidx = jax.lax.broadcasted_iota(jnp.int32, (1, B * S), 1)
    limc = jnp.zeros((B * S, 1), jnp.int32)
    limr = jnp.zeros((1, B * S), jnp.int32)
    for b in range(B):
        cnt = cnt_ref[b]
        lim = jnp.where(cnt <= S, jnp.minimum(cnt, S), S) + b * S
        inb_c = (sidx >= b * S) & (sidx < (b + 1) * S)
        limc = jnp.where(inb_c, lim, limc)
        inb_r = (lidx >= b * S) & (lidx < (b + 1) * S)
        limr = jnp.where(inb_r, lim, limr)
    validc = sidx < limc
    tokens_ref[...] = jnp.where(validc, h, 0.0)
    cents_ref[...] = jnp.where(validc, cen_ref[...], 0.0)
    valid_ref[...] = lidx < limr


def _run_mlp2(counts, tok, cen, w0, b0, w1, b1):
    full = lambda shape: pl.BlockSpec(shape, lambda: (0,) * len(shape))
    smem = pl.BlockSpec(memory_space=pltpu.SMEM)
    return pl.pallas_call(
        _mlp2_body,
        in_specs=[smem, full((B * S, TD)), full((B * S, 4)),
                  full((TD, TD)), full((1, TD)), full((TD, TD)), full((1, TD))],
        out_specs=[full((B * S, TD)), full((B * S, 4)), full((1, B * S))],
        out_shape=[jax.ShapeDtypeStruct((B * S, TD), jnp.float32),
                   jax.ShapeDtypeStruct((B * S, 4), jnp.float32),
                   jax.ShapeDtypeStruct((1, B * S), jnp.bool_)],
    )(counts, tok, cen, w0, b0.reshape(1, -1), w1, b1.reshape(1, -1))


def kernel(coords, features, batch_ids, times,
           W1_0, b1_0, W1_1, b1_1, W1_2, b1_2, W1_3, b1_3,
           W2_0, b2_0, W2_1, b2_1):
    bid = batch_ids.astype(jnp.int32)
    counts = jnp.bincount(bid, length=B).astype(jnp.int32)
    offsets = (jnp.cumsum(counts) - counts).astype(jnp.int32)
    p4 = jnp.concatenate([coords[:, :3], times], axis=1)
    xT = p4.T
    bid2 = bid.reshape(1, N)

    pf = _run_mlp1(features, [W1_0, W1_1, W1_2, W1_3], [b1_0, b1_1, b1_2, b1_3])
    cx, cy, cz, ct, gi = _run_fps(xT, p4, bid2)
    cnt16 = jnp.zeros((16,), jnp.int32).at[:B].set(counts)
    off16 = jnp.zeros((16,), jnp.int32).at[:B].set(offsets)
    knn, cenT = _sc_knn(xT, cx, cy, cz, ct, gi, cnt16, off16)
    tok = _sc_pool(pf, knn)
    tokens, centroids, valid2 = _run_mlp2(counts, tok, cenT.T,
                                          W2_0, b2_0, W2_1, b2_1)
    return (tokens.reshape(B, S, TD), centroids.reshape(B, S, 4),
            valid2.reshape(B, S))


# speedup vs baseline: 2.9157x; 1.0091x over previous
import functools

import jax
import jax.numpy as jnp
from jax import lax
from jax.experimental import pallas as pl
from jax.experimental.pallas import tpu as pltpu
from jax.experimental.pallas import tpu_sc as plsc

N = 16384
B = 8
S = 128
KNB = 16
FD = 128
TD = 768
INF = 1e10


def _gelu(x):
    return x * 0.5 * (1.0 + jax.lax.erf(x * 0.7071067811865476))


def _bdot(x, w):
    return jnp.dot(x.astype(jnp.bfloat16), w.astype(jnp.bfloat16),
                   preferred_element_type=jnp.float32)


def _mlp1_body(x_ref, w0, b0, w1, b1, w2, b2, w3, b3, o_ref):
    h = _gelu(_bdot(x_ref[...], w0[...]) + b0[...])
    h = _gelu(_bdot(h, w1[...]) + b1[...])
    h = _gelu(_bdot(h, w2[...]) + b2[...])
    o_ref[...] = _bdot(h, w3[...]) + b3[...]


def _run_mlp1(features, ws, bs):
    blk = 2048
    grid = N // blk
    full = lambda shape: pl.BlockSpec(shape, lambda i: (0,) * len(shape))
    in_specs = [pl.BlockSpec((blk, FD), lambda i: (i, 0))]
    for w, b in zip(ws, bs):
        in_specs.append(full(w.shape))
        in_specs.append(full((1,) + b.shape))
    args = [features]
    for w, b in zip(ws, bs):
        args.append(w)
        args.append(b.reshape(1, -1))
    return pl.pallas_call(
        _mlp1_body,
        grid=(grid,),
        in_specs=in_specs,
        out_specs=pl.BlockSpec((blk, TD), lambda i: (i, 0)),
        out_shape=jax.ShapeDtypeStruct((N, TD), jnp.float32),
    )(*args)


def _fps_body(xT_ref, bid_ref, cx_ref, cy_ref, cz_ref, ct_ref, gi_ref):
    xr = xT_ref[0:1, :]
    yr = xT_ref[1:2, :]
    zr = xT_ref[2:3, :]
    tr = xT_ref[3:4, :]
    bid = bid_ref[0:1, :]
    brow = jax.lax.broadcasted_iota(jnp.int32, (B, 1), 0)
    maskB = bid == brow
    gidx = jax.lax.broadcasted_iota(jnp.int32, (B, N), 1)
    lane = jax.lax.broadcasted_iota(jnp.int32, (B, S), 1)
    mind0 = jnp.where(maskB, jnp.float32(INF), jnp.float32(-INF))
    start = jnp.min(jnp.where(maskB, gidx, N), axis=1, keepdims=True)
    zf = jnp.zeros((B, S), jnp.float32)
    zi = jnp.zeros((B, S), jnp.int32)

    def body(s, carry):
        mind, cur, ax, ay, az, at, ai = carry
        onehot = gidx == cur
        cpx = jnp.sum(jnp.where(onehot, xr, 0.0), axis=1, keepdims=True)
        cpy = jnp.sum(jnp.where(onehot, yr, 0.0), axis=1, keepdims=True)
        cpz = jnp.sum(jnp.where(onehot, zr, 0.0), axis=1, keepdims=True)
        cpt = jnp.sum(jnp.where(onehot, tr, 0.0), axis=1, keepdims=True)
        sl = lane == s
        ax = ax + jnp.where(sl, cpx, 0.0)
        ay = ay + jnp.where(sl, cpy, 0.0)
        az = az + jnp.where(sl, cpz, 0.0)
        at = at + jnp.where(sl, cpt, 0.0)
        ai = ai + jnp.where(sl, cur, 0)
        dx = xr - cpx
        d = dx * dx
        dy = yr - cpy
        d = d + dy * dy
        dz = zr - cpz
        d = d + dz * dz
        dt = tr - cpt
        d = d + dt * dt
        mind = jnp.minimum(mind, d)
        m = jnp.max(mind, axis=1, keepdims=True)
        nxt = jnp.min(jnp.where(mind == m, gidx, N), axis=1, keepdims=True)
        return mind, nxt, ax, ay, az, at, ai

    _, _, ax, ay, az, at, ai = jax.lax.fori_loop(
        0, S, body, (mind0, start, zf, zf, zf, zf, zi))
    cx_ref[...] = ax
    cy_ref[...] = ay
    cz_ref[...] = az
    ct_ref[...] = at
    gi_ref[...] = ai


def _run_fps(xT, bid2):
    full = lambda shape: pl.BlockSpec(shape, lambda: (0,) * len(shape))
    outs = [jax.ShapeDtypeStruct((B, S), jnp.float32)] * 4 + [
        jax.ShapeDtypeStruct((B, S), jnp.int32)]
    return pl.pallas_call(
        _fps_body,
        in_specs=[full((4, N)), full((1, N))],
        out_specs=[full((B, S))] * 5,
        out_shape=outs,
    )(xT, bid2)


NPAD = N + 16


def _sc_scalar(ref, i):
    v = plsc.load_gather(ref, [jnp.full((16,), i, jnp.int32)])
    return v[0]


def _sc_mesh():
    return plsc.VectorSubcoreMesh(core_axis_name="c", subcore_axis_name="s",
                                  num_cores=2)


def _sc_knn(xT, cx, cy, cz, ct, gi, counts, offsets):
    @functools.partial(
        pl.kernel,
        out_type=[jax.ShapeDtypeStruct((B * S, KNB), jnp.int32),
                  jax.ShapeDtypeStruct((4, B * S), jnp.float32)],
        mesh=_sc_mesh(),
        compiler_params=pltpu.CompilerParams(needs_layout_passes=False),
        scratch_types=[
            pltpu.VMEM((4, NPAD), jnp.float32),
            pltpu.VMEM((32,), jnp.float32),
            pltpu.VMEM((32,), jnp.float32),
            pltpu.VMEM((32,), jnp.float32),
            pltpu.VMEM((32,), jnp.float32),
            pltpu.VMEM((16,), jnp.int32),
            pltpu.VMEM((16,), jnp.int32),
            pltpu.VMEM((32,), jnp.int32),
            pltpu.VMEM((32, KNB), jnp.int32),
            pltpu.VMEM((4, 32), jnp.float32),
        ],
    )
    def body(xT_h, cx_h, cy_h, cz_h, ct_h, gi_h, cnt_h, off_h,
             knn_h, cen_h,
             coords_v, cxv, cyv, czv, ctv, cntv, offv, idx32v, kbuf_v,
             cenr_v):
        cid = lax.axis_index("c")
        sid = lax.axis_index("s")
        wid = sid * 2 + cid
        b = wid // 4
        q = wid % 4
        s0 = q * 32
        lane = lax.broadcasted_iota(jnp.int32, (16,), 0)

        pltpu.sync_copy(cnt_h, cntv)
        pltpu.sync_copy(off_h, offv)
        cnt = _sc_scalar(cntv, b)
        off = _sc_scalar(offv, b)
        small = cnt <= S
        pltpu.sync_copy(xT_h, coords_v.at[:, :N])

        @pl.when(jnp.logical_not(small))
        def _():
            pltpu.sync_copy(gi_h.at[b, pl.ds(s0, 32)], idx32v)
            pltpu.sync_copy(cx_h.at[b, pl.ds(s0, 32)], cxv)
            pltpu.sync_copy(cy_h.at[b, pl.ds(s0, 32)], cyv)
            pltpu.sync_copy(cz_h.at[b, pl.ds(s0, 32)], czv)
            pltpu.sync_copy(ct_h.at[b, pl.ds(s0, 32)], ctv)
            end = off + cnt
            abase = pl.multiple_of((off // 16) * 16, 16)
            nblk = (end - abase + 15) // 16

            def tok_body(sl, _):
                cxs = _sc_scalar(cxv, sl)
                cys = _sc_scalar(cyv, sl)
                czs = _sc_scalar(czv, sl)
                cts = _sc_scalar(ctv, sl)

                def blk(i, carry):
                    tv, ti = carry
                    base = pl.multiple_of(abase + i * 16, 16)
                    dx = coords_v[0, pl.ds(base, 16)] - cxs
                    d = dx * dx
                    dy = coords_v[1, pl.ds(base, 16)] - cys
                    d = d + dy * dy
                    dz = coords_v[2, pl.ds(base, 16)] - czs
                    d = d + dz * dz
                    dt = coords_v[3, pl.ds(base, 16)] - cts
                    d = d + dt * dt
                    iv = base + lane
                    d = jnp.where((iv >= off) & (iv < end), d,
                                  jnp.float32(INF))
                    sd, si = plsc.sort_key_val(d, iv)
                    rd = lax.rev(sd, (0,))
                    ri = lax.rev(si, (0,))
                    keep_new = rd < tv
                    mv = jnp.where(keep_new, rd, tv)
                    mi = jnp.where(keep_new, ri, ti)
                    return tuple(plsc.sort_key_val(mv, mi))

                tv0 = jnp.full((16,), jnp.float32(INF))
                ti0 = jnp.zeros((16,), jnp.int32)
                _, ti = lax.fori_loop(0, nblk, blk, (tv0, ti0))
                kbuf_v[sl, :] = ti
                return 0

            lax.fori_loop(0, 32, tok_body, 0)

        @pl.when(small)
        def _():
            base = off + s0
            idx32v[pl.ds(0, 16)] = jnp.minimum(base + lane, N - 1)
            idx32v[pl.ds(16, 16)] = jnp.minimum(base + 16 + lane, N - 1)

            def tok_body(sl, _):
                row = plsc.load_gather(
                    idx32v, [jnp.full((16,), sl, jnp.int32)])
                kbuf_v[sl, :] = row
                return 0

            lax.fori_loop(0, 32, tok_body, 0)

        t0 = b * S + s0
        for h in (0, 16):
            idx16 = idx32v[pl.ds(h, 16)]
            for c in range(4):
                g = plsc.load_gather(coords_v,
                                     [jnp.full((16,), c, jnp.int32), idx16])
                cenr_v[c, pl.ds(h, 16)] = g
        for c in range(4):
            pltpu.sync_copy(cenr_v.at[c, pl.ds(0, 32)],
                            cen_h.at[c, pl.ds(t0, 32)])
        pltpu.sync_copy(kbuf_v, knn_h.at[pl.ds(t0, 32), :])

    return body(xT, cx, cy, cz, ct, gi, counts, offsets)


def _sc_pool(pf, knn):
    @functools.partial(
        pl.kernel,
        out_type=jax.ShapeDtypeStruct((B * S, TD), jnp.float32),
        mesh=_sc_mesh(),
        compiler_params=pltpu.CompilerParams(needs_layout_passes=False),
        scratch_types=[
            pltpu.VMEM((32, KNB), jnp.int32),
            pltpu.VMEM((16,), jnp.int32),
            pltpu.VMEM((16,), jnp.int32),
            pltpu.VMEM((KNB, TD), jnp.float32),
            pltpu.VMEM((KNB, TD), jnp.float32),
            pltpu.VMEM((32, TD), jnp.float32),
            pltpu.SemaphoreType.DMA,
            pltpu.SemaphoreType.DMA,
        ],
    )
    def body(pf_h, knn_h, tok_h, kbuf_v, idx0, idx1, rows0, rows1, out_v,
             sem0, sem1):
        cid = lax.axis_index("c")
        sid = lax.axis_index("s")
        wid = sid * 2 + cid
        t0 = wid * 32
        pltpu.sync_copy(knn_h.at[pl.ds(t0, 32), :], kbuf_v)

        def pool(rows_v, sl):
            def step(c, _):
                cb = pl.multiple_of(c * 16, 16)
                acc = rows_v[0, pl.ds(cb, 16)]
                for r in range(1, KNB):
                    acc = jnp.maximum(acc, rows_v[r, pl.ds(cb, 16)])
                out_v[sl, pl.ds(cb, 16)] = acc
                return 0
            lax.fori_loop(0, TD // 16, step, 0)

        idx0[...] = kbuf_v[0, :]
        pltpu.async_copy(pf_h.at[idx0], rows0, sem0)

        def pair(p, _):
            ta = 2 * p
            idx1[...] = kbuf_v[ta + 1, :]
            pltpu.async_copy(pf_h.at[idx1], rows1, sem1)
            pltpu.make_async_copy(pf_h.at[idx0], rows0, sem0).wait()
            pool(rows0, ta)

            @pl.when(p < 15)
            def _():
                idx0[...] = kbuf_v[ta + 2, :]
                pltpu.async_copy(pf_h.at[idx0], rows0, sem0)

            pltpu.make_async_copy(pf_h.at[idx1], rows1, sem1).wait()
            pool(rows1, ta + 1)
            return 0

        lax.fori_loop(0, 16, pair, 0)
        pltpu.sync_copy(out_v, tok_h.at[pl.ds(t0, 32), :])

    return body(pf, knn)


def _mlp2_body(cnt_ref, tok_ref, cen_ref, w0, b0, w1, b1,
               tokens_ref, cents_ref, valid_ref):
    t = tok_ref[...]
    h = _gelu(_bdot(t, w0[...]) + b0[...])
    h = _bdot(h, w1[...]) + b1[...]
    sidx = jax.lax.broadcasted_iota(jnp.int32, (B * S, 1), 0)
    lidx = jax.lax.broadcasted_iota(jnp.int32, (1, B * S), 1)
    limc = jnp.zeros((B * S, 1), jnp.int32)
    limr = jnp.zeros((1, B * S), jnp.int32)
    for b in range(B):
        cnt = cnt_ref[b]
        lim = jnp.where(cnt <= S, jnp.minimum(cnt, S), S) + b * S
        inb_c = (sidx >= b * S) & (sidx < (b + 1) * S)
        limc = jnp.where(inb_c, lim, limc)
        inb_r = (lidx >= b * S) & (lidx < (b + 1) * S)
        limr = jnp.where(inb_r, lim, limr)
    validc = sidx < limc
    tokens_ref[...] = jnp.where(validc, h, 0.0)
    cents_ref[...] = jnp.where(validc, cen_ref[...], 0.0)
    valid_ref[...] = lidx < limr


def _run_mlp2(counts, tok, cen, w0, b0, w1, b1):
    full = lambda shape: pl.BlockSpec(shape, lambda: (0,) * len(shape))
    smem = pl.BlockSpec(memory_space=pltpu.SMEM)
    return pl.pallas_call(
        _mlp2_body,
        in_specs=[smem, full((B * S, TD)), full((B * S, 4)),
                  full((TD, TD)), full((1, TD)), full((TD, TD)), full((1, TD))],
        out_specs=[full((B * S, TD)), full((B * S, 4)), full((1, B * S))],
        out_shape=[jax.ShapeDtypeStruct((B * S, TD), jnp.float32),
                   jax.ShapeDtypeStruct((B * S, 4), jnp.float32),
                   jax.ShapeDtypeStruct((1, B * S), jnp.bool_)],
    )(counts, tok, cen, w0, b0.reshape(1, -1), w1, b1.reshape(1, -1))


def kernel(coords, features, batch_ids, times,
           W1_0, b1_0, W1_1, b1_1, W1_2, b1_2, W1_3, b1_3,
           W2_0, b2_0, W2_1, b2_1):
    bid = batch_ids.astype(jnp.int32)
    counts = jnp.bincount(bid, length=B).astype(jnp.int32)
    offsets = (jnp.cumsum(counts) - counts).astype(jnp.int32)
    p4 = jnp.concatenate([coords[:, :3], times], axis=1)
    xT = p4.T
    bid2 = bid.reshape(1, N)

    pf = _run_mlp1(features, [W1_0, W1_1, W1_2, W1_3], [b1_0, b1_1, b1_2, b1_3])
    cx, cy, cz, ct, gi = _run_fps(xT, bid2)
    cnt16 = jnp.zeros((16,), jnp.int32).at[:B].set(counts)
    off16 = jnp.zeros((16,), jnp.int32).at[:B].set(offsets)
    knn, cenT = _sc_knn(xT, cx, cy, cz, ct, gi, cnt16, off16)
    tok = _sc_pool(pf, knn)
    tokens, centroids, valid2 = _run_mlp2(counts, tok, cenT.T,
                                          W2_0, b2_0, W2_1, b2_1)
    return (tokens.reshape(B, S, TD), centroids.reshape(B, S, 4),
            valid2.reshape(B, S))
